# Initial kernel scaffold; baseline (speedup 1.0000x reference)
import jax
import jax.numpy as jnp
from jax.experimental import pallas as pl


def kernel(x, edge_index, batch, gamma1, beta1, W1, att_src1, att_dst1, bias1, gamma2, beta2, W2, att_src2, att_dst2, bias2, Wf, bf):
    raise NotImplementedError("write your pallas kernel here")


# speedup vs baseline: 1.0005x; 1.0005x over previous
import jax
import jax.numpy as jnp
from jax.experimental import pallas as pl

B = 4
N = 10000
E = 320000
D = 128
H = 4
F = 64
C = 4
G = 64


def _batchnorm(x, gamma, beta):
    mu = x.mean(axis=0)
    var = x.var(axis=0)
    return (x - mu) / jnp.sqrt(var + 1e-5) * gamma + beta


def _gat_conv(x, src, dst, W, att_src, att_dst, bias, heads, out_ch, concat):
    n = x.shape[0]
    h = (x @ W).reshape(n, heads, out_ch)
    a_src = (h * att_src[None]).sum(-1)
    a_dst = (h * att_dst[None]).sum(-1)
    e = jax.nn.leaky_relu(a_src[src] + a_dst[dst], 0.2)
    emax = jax.ops.segment_max(e, dst, num_segments=n)
    emax = jnp.where(jnp.isfinite(emax), emax, 0.0)
    ex = jnp.exp(e - jax.lax.stop_gradient(emax)[dst])
    denom = jax.ops.segment_sum(ex, dst, num_segments=n)
    alpha = ex / (denom[dst] + 1e-16)
    msg = h[src] * alpha[:, :, None]
    out = jax.ops.segment_sum(msg, dst, num_segments=n)
    if concat:
        out = out.reshape(n, heads * out_ch)
    else:
        out = out.mean(axis=1)
    return out + bias


def _gat_net(x, ei, batch, p):
    loops = jnp.arange(N)
    src = jnp.concatenate([ei[0], loops])
    dst = jnp.concatenate([ei[1], loops])
    h = _batchnorm(x, p["gamma1"], p["beta1"])
    h = _gat_conv(h, src, dst, p["W1"], p["att_src1"], p["att_dst1"], p["bias1"], H, F, True)
    h = jax.nn.relu(h)
    h = _batchnorm(h, p["gamma2"], p["beta2"])
    h = _gat_conv(h, src, dst, p["W2"], p["att_src2"], p["att_dst2"], p["bias2"], 1, C, False)
    h = jax.nn.gelu(h, approximate=False)
    sums = jax.ops.segment_sum(h, batch, num_segments=G)
    cnts = jax.ops.segment_sum(jnp.ones((N, 1), jnp.float32), batch, num_segments=G)
    pooled = sums / jnp.maximum(cnts, 1.0)
    return jax.nn.log_softmax(pooled, axis=1)


def _fusion_kernel(xc_ref, wf_ref, bf_ref, out_ref):
    out_ref[...] = jax.nn.relu(
        jnp.dot(xc_ref[...], wf_ref[...], preferred_element_type=jnp.float32)
        + bf_ref[...]
    )


def kernel(x, edge_index, batch, gamma1, beta1, W1, att_src1, att_dst1, bias1,
           gamma2, beta2, W2, att_src2, att_dst2, bias2, Wf, bf):
    outs = []
    for b in range(B):
        p = {"gamma1": gamma1[b], "beta1": beta1[b], "W1": W1[b],
             "att_src1": att_src1[b], "att_dst1": att_dst1[b], "bias1": bias1[b],
             "gamma2": gamma2[b], "beta2": beta2[b], "W2": W2[b],
             "att_src2": att_src2[b], "att_dst2": att_dst2[b], "bias2": bias2[b]}
        outs.append(_gat_net(x[b], edge_index[b], batch, p))
    xc = jnp.concatenate(outs, axis=1)
    return pl.pallas_call(
        _fusion_kernel,
        out_shape=jax.ShapeDtypeStruct((G, C), jnp.float32),
    )(xc, Wf, bf[None, :])


# speedup vs baseline: 16.7734x; 16.7648x over previous
import functools

import jax
import jax.numpy as jnp
from jax import lax
from jax.experimental import pallas as pl
from jax.experimental.pallas import tpu as pltpu
from jax.experimental.pallas import tpu_sc as plsc

B = 4
N = 10000
E = 320000
D = 128
H = 4
F = 64
C = 4
G = 64

RT = 1000
NT = N // RT
CH = 80
EPT16 = E // 16
EPT32 = E // 32
NP = 10240
NPT = NP // 16
NH1 = 3456
NP1 = 3 * NH1
NH1T = NH1 // 16
NH = 5120
NHT = NH // 16


def _leaky(x):
    return jnp.where(x >= 0, x, 0.2 * x)


def _stats_body(x_ref, s_ref, q_ref):
    nt = pl.program_id(1)
    xb = x_ref[0]
    s = jnp.sum(xb, axis=0, keepdims=True)
    q = jnp.sum(xb * xb, axis=0, keepdims=True)

    @pl.when(nt == 0)
    def _():
        s_ref[0] = s
        q_ref[0] = q

    @pl.when(nt != 0)
    def _():
        s_ref[0] = s_ref[0] + s
        q_ref[0] = q_ref[0] + q


def _tc_stats(x, d):
    return pl.pallas_call(
        _stats_body,
        grid=(B, NT),
        in_specs=[pl.BlockSpec((1, RT, d), lambda b, nt: (b, nt, 0))],
        out_specs=[pl.BlockSpec((1, 1, d), lambda b, nt: (b, 0, 0)),
                   pl.BlockSpec((1, 1, d), lambda b, nt: (b, 0, 0))],
        out_shape=[jax.ShapeDtypeStruct((B, 1, d), jnp.float32),
                   jax.ShapeDtypeStruct((B, 1, d), jnp.float32)],
    )(x)


def _mm1_body(x_ref, w_ref, wa_ref, c1_ref, ca_ref,
              hp_ref, a1_ref, gmax_ref):
    nt = pl.program_id(1)
    xb = x_ref[0]
    h = jnp.dot(xb, w_ref[0], preferred_element_type=jnp.float32) + c1_ref[0]
    a = jnp.dot(xb, wa_ref[0], preferred_element_type=jnp.float32) + ca_ref[0]
    hp_ref[0, 0] = h[:, :128]
    hp_ref[0, 1] = h[:, 128:]
    a1_ref[0] = a
    cm = jnp.max(a, axis=0, keepdims=True)

    @pl.when(nt == 0)
    def _():
        gmax_ref[0] = cm

    @pl.when(nt != 0)
    def _():
        gmax_ref[0] = jnp.maximum(gmax_ref[0], cm)


def _tc_mm1(x, W1e, Wa, c1, ca):
    return pl.pallas_call(
        _mm1_body,
        grid=(B, NT),
        in_specs=[pl.BlockSpec((1, RT, D), lambda b, nt: (b, nt, 0)),
                  pl.BlockSpec((1, D, 256), lambda b, nt: (b, 0, 0)),
                  pl.BlockSpec((1, D, 8), lambda b, nt: (b, 0, 0)),
                  pl.BlockSpec((1, 1, 256), lambda b, nt: (b, 0, 0)),
                  pl.BlockSpec((1, 1, 8), lambda b, nt: (b, 0, 0))],
        out_specs=[pl.BlockSpec((1, 2, RT, 128), lambda b, nt: (b, 0, nt, 0)),
                   pl.BlockSpec((1, RT, 8), lambda b, nt: (b, nt, 0)),
                   pl.BlockSpec((1, 1, 8), lambda b, nt: (b, 0, 0))],
        out_shape=[jax.ShapeDtypeStruct((B, 2, N, 128), jnp.float32),
                   jax.ShapeDtypeStruct((B, N, 8), jnp.float32),
                   jax.ShapeDtypeStruct((B, 1, 8), jnp.float32)],
    )(x, W1e, Wa, c1, ca)


def _norm1_body(msg_ref, den_ref, hp_ref, a1_ref, gmax_ref, b1_ref,
                xr_ref, s2_ref, q2_ref):
    nt = pl.program_id(1)
    a = a1_ref[0]
    asrc = a[:, 0:4]
    adst = a[:, 4:8]
    gm = gmax_ref[0, 0, 0:4]
    shift = _leaky(gm[None, :] + adst)
    wself = jnp.exp(_leaky(asrc + adst) - shift)
    den4 = jnp.concatenate([den_ref[0, 0], den_ref[0, 1]], axis=1) + wself
    dinv = 1.0 / (den4 + 1e-16)
    for h in range(4):
        half = h // 2
        col = (h % 2) * 64
        msg = msg_ref[0, half][:, col:col + 64]
        hcols = hp_ref[0, half][:, col:col + 64]
        o = (msg + wself[:, h:h + 1] * hcols) * dinv[:, h:h + 1] \
            + b1_ref[0, 0, h * 64:(h + 1) * 64][None, :]
        xr = jnp.maximum(o, 0.0)
        xr_ref[0, :, h * 64:(h + 1) * 64] = xr
        s = jnp.sum(xr, axis=0, keepdims=True)
        q = jnp.sum(xr * xr, axis=0, keepdims=True)

        @pl.when(nt == 0)
        def _():
            s2_ref[0, :, h * 64:(h + 1) * 64] = s
            q2_ref[0, :, h * 64:(h + 1) * 64] = q

        @pl.when(nt != 0)
        def _():
            s2_ref[0, :, h * 64:(h + 1) * 64] = s2_ref[0, :, h * 64:(h + 1) * 64] + s
            q2_ref[0, :, h * 64:(h + 1) * 64] = q2_ref[0, :, h * 64:(h + 1) * 64] + q


def _tc_norm1(msgacc, den, hp, a1, gmax, bias1):
    return pl.pallas_call(
        _norm1_body,
        grid=(B, NT),
        in_specs=[pl.BlockSpec((1, 2, RT, 128), lambda b, nt: (b, 0, nt, 0)),
                  pl.BlockSpec((1, 2, RT, 2), lambda b, nt: (b, 0, nt, 0)),
                  pl.BlockSpec((1, 2, RT, 128), lambda b, nt: (b, 0, nt, 0)),
                  pl.BlockSpec((1, RT, 8), lambda b, nt: (b, nt, 0)),
                  pl.BlockSpec((1, 1, 8), lambda b, nt: (b, 0, 0)),
                  pl.BlockSpec((1, 1, 256), lambda b, nt: (b, 0, 0))],
        out_specs=[pl.BlockSpec((1, RT, 256), lambda b, nt: (b, nt, 0)),
                   pl.BlockSpec((1, 1, 256), lambda b, nt: (b, 0, 0)),
                   pl.BlockSpec((1, 1, 256), lambda b, nt: (b, 0, 0))],
        out_shape=[jax.ShapeDtypeStruct((B, N, 256), jnp.float32),
                   jax.ShapeDtypeStruct((B, 1, 256), jnp.float32),
                   jax.ShapeDtypeStruct((B, 1, 256), jnp.float32)],
    )(msgacc, den, hp, a1, gmax, bias1)


def _mm2_body(xr_ref, p2_ref, c2_ref, tab_ref, cmax_ref):
    nt = pl.program_id(1)
    t = jnp.dot(xr_ref[0], p2_ref[0], preferred_element_type=jnp.float32) + c2_ref[0]
    tab_ref[0] = t
    cm = jnp.max(t, axis=0, keepdims=True)

    @pl.when(nt == 0)
    def _():
        cmax_ref[0] = cm

    @pl.when(nt != 0)
    def _():
        cmax_ref[0] = jnp.maximum(cmax_ref[0], cm)


def _tc_mm2(xr, P2, c2):
    return pl.pallas_call(
        _mm2_body,
        grid=(B, NT),
        in_specs=[pl.BlockSpec((1, RT, 256), lambda b, nt: (b, nt, 0)),
                  pl.BlockSpec((1, 256, 8), lambda b, nt: (b, 0, 0)),
                  pl.BlockSpec((1, 1, 8), lambda b, nt: (b, 0, 0))],
        out_specs=[pl.BlockSpec((1, RT, 8), lambda b, nt: (b, nt, 0)),
                   pl.BlockSpec((1, 1, 8), lambda b, nt: (b, 0, 0))],
        out_shape=[jax.ShapeDtypeStruct((B, N, 8), jnp.float32),
                   jax.ShapeDtypeStruct((B, 1, 8), jnp.float32)],
    )(xr, P2, c2)


def _norm2_body(acc_ref, tab_ref, gmax_ref, b2_ref, batch_ref,
                ps_ref, pc_ref):
    nt = pl.program_id(1)
    th = tab_ref[0]
    h2 = th[:, 0:4]
    as2 = th[:, 4:5]
    ad2 = th[:, 5:6]
    g2 = gmax_ref[0, 0, 4]
    wself = jnp.exp(_leaky(as2 + ad2) - _leaky(g2 + ad2))
    a0 = acc_ref[0, 0]
    a1_ = acc_ref[1, 0]
    den = a0[:, 4:5] + a1_[:, 4:5] + wself
    msg = a0[:, 0:4] + a1_[:, 0:4] + wself * h2
    o2 = msg / (den + 1e-16) + b2_ref[0, 0][None, :]
    gel = 0.5 * o2 * (1.0 + lax.erf(o2 * 0.7071067811865476))
    oh = (batch_ref[:, 0:1] == lax.broadcasted_iota(jnp.int32, (1, G), 1)
          ).astype(jnp.float32)
    psum = lax.dot_general(oh, gel, (((0,), (0,)), ((), ())),
                           preferred_element_type=jnp.float32)
    ones = jnp.ones((RT, 1), jnp.float32)
    pcnt = lax.dot_general(oh, ones, (((0,), (0,)), ((), ())),
                           preferred_element_type=jnp.float32)

    @pl.when(nt == 0)
    def _():
        ps_ref[0] = psum
        pc_ref[0] = pcnt

    @pl.when(nt != 0)
    def _():
        ps_ref[0] = ps_ref[0] + psum
        pc_ref[0] = pc_ref[0] + pcnt


def _tc_norm2pool(acc2, tab2, gmax2, bias2, batch2d):
    return pl.pallas_call(
        _norm2_body,
        grid=(B, NT),
        in_specs=[pl.BlockSpec((2, 1, RT, 8), lambda b, nt: (0, b, nt, 0)),
                  pl.BlockSpec((1, RT, 8), lambda b, nt: (b, nt, 0)),
                  pl.BlockSpec((1, 1, 8), lambda b, nt: (b, 0, 0)),
                  pl.BlockSpec((1, 1, 4), lambda b, nt: (b, 0, 0)),
                  pl.BlockSpec((RT, 1), lambda b, nt: (nt, 0))],
        out_specs=[pl.BlockSpec((1, G, 4), lambda b, nt: (b, 0, 0)),
                   pl.BlockSpec((1, G, 1), lambda b, nt: (b, 0, 0))],
        out_shape=[jax.ShapeDtypeStruct((B, G, 4), jnp.float32),
                   jax.ShapeDtypeStruct((B, G, 1), jnp.float32)],
    )(acc2, tab2, gmax2, bias2, batch2d)


def _fin_body(ps_ref, pc_ref, wf_ref, bf_ref, out_ref):
    acc = jnp.zeros((G, C), jnp.float32)
    for b in range(B):
        pooled = ps_ref[b] / jnp.maximum(pc_ref[b], 1.0)
        m = jnp.max(pooled, axis=1, keepdims=True)
        z = pooled - m
        ls = z - jnp.log(jnp.sum(jnp.exp(z), axis=1, keepdims=True))
        acc = acc + jnp.dot(ls, wf_ref[4 * b:4 * b + 4, :],
                            preferred_element_type=jnp.float32)
    out_ref[...] = jnp.maximum(acc + bf_ref[0][None, :], 0.0)


def _tc_fin(psum, pcnt, Wf, bf):
    return pl.pallas_call(
        _fin_body,
        out_shape=jax.ShapeDtypeStruct((G, C), jnp.float32),
    )(psum, pcnt, Wf, bf[None, :])


def _sc1_body(a1_h, gmax_h, src_h, dst_h, hp_h,
              msg_h, den_h,
              tab_v, gbuf_v, idxs_v, idxd_v, idxl_v, iw_v, wab_v, wden_v,
              rows_v, zrow_v, zbuf_v, acc_s, dacc_s, sem):
    c = lax.axis_index("c")
    sid = lax.axis_index("s")

    def zr(r, _):
        for q in range(8):
            zrow_v[r, pl.ds(q * 16, 16)] = jnp.zeros((16,), jnp.float32)
        return 0

    lax.fori_loop(0, 72, zr, 0)

    def zb(i, _):
        zbuf_v[pl.ds(i * 16, 16)] = jnp.zeros((16,), jnp.float32)
        return 0

    lax.fori_loop(0, 80, zb, 0)

    for b in range(B):
        pltpu.sync_copy(zbuf_v, dacc_s.at[pl.ds(sid * 1280, 1280)])
        pltpu.sync_copy(a1_h.at[b, 0], tab_v)
        pltpu.sync_copy(gmax_h.at[b, 0], gbuf_v)
        for p in range(3):
            lo = p * NH1
            for i in range(3):
                pltpu.sync_copy(zrow_v,
                                acc_s.at[pl.ds(sid * NH1T + i * 72, 72)])
            plsc.subcore_barrier()

            def chunk(k, _):
                base = b * E + sid * EPT16 + k * CH
                pltpu.sync_copy(src_h.at[pl.ds(base, CH)], idxs_v)
                pltpu.sync_copy(dst_h.at[pl.ds(base, CH)], idxd_v)
                for j in range(CH // 16):
                    sv = idxs_v[pl.ds(j * 16, 16)]
                    dv = idxd_v[pl.ds(j * 16, 16)]
                    inr = (dv >= lo) & (dv < lo + NH1)
                    idxl_v[pl.ds(j * 16, 16)] = jnp.where(inr, dv - lo, 0)
                    for hh in range(2):
                        hvec = jnp.broadcast_to(c * 2 + hh, (16,)).astype(jnp.int32)
                        sa = plsc.load_gather(tab_v, [sv * 8 + hvec])
                        da = plsc.load_gather(tab_v, [dv * 8 + hvec + 4])
                        gm = plsc.load_gather(gbuf_v, [hvec])
                        w = jnp.exp(_leaky(sa + da) - _leaky(gm + da))
                        wab_v[pl.ds(hh * CH + j * 16, 16)] = jnp.where(inr, w, 0.0)
                        if p == 0:
                            wden_v[pl.ds(hh * CH + j * 16, 16)] = w
                            iw_v[hh, pl.ds(j * 16, 16)] = dv * 2 + hh

                pltpu.async_copy(hp_h.at[b, c].at[idxs_v], rows_v, sem).wait()

                def sbody(e, _):
                    e16 = jnp.broadcast_to(e, (16,)).astype(jnp.int32)
                    wa = plsc.load_gather(wab_v, [e16])
                    wb = plsc.load_gather(wab_v, [e16 + CH])
                    for q in range(4):
                        rows_v[e, pl.ds(q * 16, 16)] = rows_v[e, pl.ds(q * 16, 16)] * wa
                    for q in range(4, 8):
                        rows_v[e, pl.ds(q * 16, 16)] = rows_v[e, pl.ds(q * 16, 16)] * wb
                    return 0

                lax.fori_loop(0, CH, sbody, 0)
                pltpu.sync_copy(rows_v, acc_s.at[idxl_v], add=True)
                if p == 0:
                    pltpu.sync_copy(wden_v.at[pl.ds(0, CH)],
                                    dacc_s.at[iw_v.at[0]], add=True)
                    pltpu.sync_copy(wden_v.at[pl.ds(CH, CH)],
                                    dacc_s.at[iw_v.at[1]], add=True)
                return 0

            lax.fori_loop(0, EPT16 // CH, chunk, 0)
            plsc.subcore_barrier()
            pltpu.sync_copy(acc_s.at[pl.ds(sid * NH1T, NH1T)],
                            msg_h.at[b, c, p, pl.ds(sid * NH1T, NH1T)])

        @pl.when(sid == 0)
        def _():
            pltpu.sync_copy(dacc_s, den_h.at[b, c, 0])

        plsc.subcore_barrier()


def _sc_layer1(a1, gmax16, src_flat, dst_flat, hp):
    mesh = plsc.VectorSubcoreMesh(core_axis_name="c", subcore_axis_name="s")
    f = pl.kernel(
        _sc1_body,
        mesh=mesh,
        compiler_params=pltpu.CompilerParams(needs_layout_passes=False),
        out_type=[jax.ShapeDtypeStruct((B, 2, 3, NH1, 128), jnp.float32),
                  jax.ShapeDtypeStruct((B, 2, 1, 20480), jnp.float32)],
        scratch_types=[
            pltpu.VMEM((N * 8,), jnp.float32),
            pltpu.VMEM((16,), jnp.float32),
            pltpu.VMEM((CH,), jnp.int32),
            pltpu.VMEM((CH,), jnp.int32),
            pltpu.VMEM((CH,), jnp.int32),
            pltpu.VMEM((2, CH), jnp.int32),
            pltpu.VMEM((2 * CH,), jnp.float32),
            pltpu.VMEM((2 * CH,), jnp.float32),
            pltpu.VMEM((CH, 128), jnp.float32),
            pltpu.VMEM((72, 128), jnp.float32),
            pltpu.VMEM((1280,), jnp.float32),
            pltpu.VMEM_SHARED((NH1, 128), jnp.float32),
            pltpu.VMEM_SHARED((20480,), jnp.float32),
            pltpu.SemaphoreType.DMA,
        ],
    )
    return f(a1, gmax16, src_flat, dst_flat, hp)


def _sc2_body(tab_h, gmax_h, src_h, dst_h,
              acc_h,
              tab_v, gbuf_v, idxs_v, idxd_v, i8_v, val_v, zbuf_v, acc8_s):
    c = lax.axis_index("c")
    sid = lax.axis_index("s")
    four16 = jnp.full((16,), 4, jnp.int32)
    five16 = jnp.full((16,), 5, jnp.int32)

    def zb(i, _):
        zbuf_v[pl.ds(i * 16, 16)] = jnp.zeros((16,), jnp.float32)
        return 0

    lax.fori_loop(0, 80, zb, 0)

    for b in range(B):
        pltpu.sync_copy(tab_h.at[b, 0], tab_v)
        pltpu.sync_copy(gmax_h.at[b, 0], gbuf_v)
        for p in range(2):
            lo = p * NH
            for i in range(2):
                pltpu.sync_copy(zbuf_v,
                                acc8_s.at[pl.ds(sid * 2560 + i * 1280, 1280)])
            plsc.subcore_barrier()

            def chunk(k, _):
                base = b * E + (sid * 2 + c) * EPT32 + k * CH
                pltpu.sync_copy(src_h.at[pl.ds(base, CH)], idxs_v)
                pltpu.sync_copy(dst_h.at[pl.ds(base, CH)], idxd_v)
                for j in range(CH // 16):
                    sv = idxs_v[pl.ds(j * 16, 16)]
                    dv = idxd_v[pl.ds(j * 16, 16)]
                    inr = (dv >= lo) & (dv < lo + NH)
                    dvl = jnp.where(inr, dv - lo, 0)
                    as2 = plsc.load_gather(tab_v, [sv * 8 + four16])
                    ad2 = plsc.load_gather(tab_v, [dv * 8 + five16])
                    g2 = plsc.load_gather(gbuf_v, [four16])
                    w = jnp.exp(_leaky(as2 + ad2) - _leaky(g2 + ad2))
                    w = jnp.where(inr, w, 0.0)
                    val_v[4, pl.ds(j * 16, 16)] = w
                    i8_v[4, pl.ds(j * 16, 16)] = dvl * 8 + 4
                    for cc in range(4):
                        cc16 = jnp.full((16,), cc, jnp.int32)
                        hcc = plsc.load_gather(tab_v, [sv * 8 + cc16])
                        val_v[cc, pl.ds(j * 16, 16)] = w * hcc
                        i8_v[cc, pl.ds(j * 16, 16)] = dvl * 8 + cc
                for cc in range(5):
                    pltpu.sync_copy(val_v.at[cc], acc8_s.at[i8_v.at[cc]], add=True)
                return 0

            lax.fori_loop(0, EPT32 // CH, chunk, 0)
            plsc.subcore_barrier()

            @pl.when(sid == 0)
            def _():
                pltpu.sync_copy(acc8_s, acc_h.at[b, c, p])

            plsc.subcore_barrier()


def _sc_layer2(tab2, gmax16, src_flat, dst_flat):
    mesh = plsc.VectorSubcoreMesh(core_axis_name="c", subcore_axis_name="s")
    f = pl.kernel(
        _sc2_body,
        mesh=mesh,
        compiler_params=pltpu.CompilerParams(needs_layout_passes=False),
        out_type=[jax.ShapeDtypeStruct((B, 2, 2, 8 * NH), jnp.float32)],
        scratch_types=[
            pltpu.VMEM((N * 8,), jnp.float32),
            pltpu.VMEM((16,), jnp.float32),
            pltpu.VMEM((CH,), jnp.int32),
            pltpu.VMEM((CH,), jnp.int32),
            pltpu.VMEM((5, CH), jnp.int32),
            pltpu.VMEM((5, CH), jnp.float32),
            pltpu.VMEM((1280,), jnp.float32),
            pltpu.VMEM_SHARED((8 * NH,), jnp.float32),
        ],
    )
    return f(tab2, gmax16, src_flat, dst_flat)[0]


def kernel(x, edge_index, batch, gamma1, beta1, W1, att_src1, att_dst1, bias1,
           gamma2, beta2, W2, att_src2, att_dst2, bias2, Wf, bf):
    ei = edge_index.astype(jnp.int32)

    s0, q0 = _tc_stats(x, D)
    mu = s0[:, 0, :] / N
    var = q0[:, 0, :] / N - mu * mu
    sc1 = gamma1 / jnp.sqrt(var + 1e-5)
    sh1 = beta1 - mu * sc1
    W1e = W1 * sc1[:, :, None]
    c1 = jnp.einsum("bd,bdk->bk", sh1, W1)[:, None, :]
    m4 = (jnp.arange(256)[:, None] // 64 == jnp.arange(4)[None, :]).astype(jnp.float32)
    A8 = jnp.concatenate([att_src1.reshape(B, 256, 1) * m4[None],
                          att_dst1.reshape(B, 256, 1) * m4[None]], axis=2)
    Wa = jnp.einsum("bdk,bkh->bdh", W1e, A8)
    ca = jnp.einsum("bk,bkh->bh", c1[:, 0, :], A8)[:, None, :]

    hp, a1, gmax1 = _tc_mm1(x, W1e, Wa, c1, ca)
    g16 = jnp.pad(gmax1, ((0, 0), (0, 0), (0, 8)))
    src_flat = ei[:, 0, :].reshape(B * E)
    dst_flat = ei[:, 1, :].reshape(B * E)

    a1f = a1.reshape(B, 1, N * 8)
    msgacc, denflat = _sc_layer1(a1f, g16, src_flat, dst_flat, hp)
    msgacc = msgacc.reshape(B, 2, NP1, 128)
    den = denflat[:, :, 0, :2 * N].reshape(B, 2, N, 2)

    xr, s2, q2 = _tc_norm1(msgacc, den, hp, a1, gmax1, bias1[:, None, :])
    mu2 = s2[:, 0, :] / N
    var2 = q2[:, 0, :] / N - mu2 * mu2
    sc2 = gamma2 / jnp.sqrt(var2 + 1e-5)
    sh2 = beta2 - mu2 * sc2
    W2e = W2 * sc2[:, :, None]
    c2 = jnp.einsum("bd,bdk->bk", sh2, W2)
    as2v = att_src2[:, 0, :]
    ad2v = att_dst2[:, 0, :]
    P2 = jnp.concatenate([
        W2e,
        jnp.einsum("bdk,bk->bd", W2e, as2v)[:, :, None],
        jnp.einsum("bdk,bk->bd", W2e, ad2v)[:, :, None],
        jnp.zeros((B, 256, 2), jnp.float32),
    ], axis=2)
    c2cat = jnp.concatenate([
        c2,
        jnp.einsum("bk,bk->b", c2, as2v)[:, None],
        jnp.einsum("bk,bk->b", c2, ad2v)[:, None],
        jnp.zeros((B, 2), jnp.float32),
    ], axis=1)[:, None, :]

    tab2, cmax2 = _tc_mm2(xr, P2, c2cat)
    g216 = jnp.pad(cmax2, ((0, 0), (0, 0), (0, 8)))

    tab2f = tab2.reshape(B, 1, N * 8)
    acc2 = _sc_layer2(tab2f, g216, src_flat, dst_flat).reshape(B, 2, NP, 8)
    acc2 = jnp.transpose(acc2, (1, 0, 2, 3))

    psum, pcnt = _tc_norm2pool(acc2, tab2, cmax2, bias2[:, None, :],
                               batch.astype(jnp.int32)[:, None])

    return _tc_fin(psum, pcnt, Wf, bf)


# speedup vs baseline: 21.6264x; 1.2893x over previous
import functools

import jax
import jax.numpy as jnp
from jax import lax
from jax.experimental import pallas as pl
from jax.experimental.pallas import tpu as pltpu
from jax.experimental.pallas import tpu_sc as plsc

B = 4
N = 10000
E = 320000
D = 128
H = 4
F = 64
C = 4
G = 64

RT = 1000
NT = N // RT
CH = 80
CH1 = 128
EPTF = 19968
NCH1 = EPTF // CH1
EPT16 = E // 16
EPT32 = E // 32
NP = 10240
NPT = NP // 16
NH1 = 3456
NP1 = 3 * NH1
NH1T = NH1 // 16
NH = 5120
NHT = NH // 16


def _leaky(x):
    return jnp.where(x >= 0, x, 0.2 * x)


def _stats_body(x_ref, s_ref, q_ref):
    nt = pl.program_id(1)
    xb = x_ref[0]
    s = jnp.sum(xb, axis=0, keepdims=True)
    q = jnp.sum(xb * xb, axis=0, keepdims=True)

    @pl.when(nt == 0)
    def _():
        s_ref[0] = s
        q_ref[0] = q

    @pl.when(nt != 0)
    def _():
        s_ref[0] = s_ref[0] + s
        q_ref[0] = q_ref[0] + q


def _tc_stats(x, d):
    return pl.pallas_call(
        _stats_body,
        grid=(B, NT),
        in_specs=[pl.BlockSpec((1, RT, d), lambda b, nt: (b, nt, 0))],
        out_specs=[pl.BlockSpec((1, 1, d), lambda b, nt: (b, 0, 0)),
                   pl.BlockSpec((1, 1, d), lambda b, nt: (b, 0, 0))],
        out_shape=[jax.ShapeDtypeStruct((B, 1, d), jnp.float32),
                   jax.ShapeDtypeStruct((B, 1, d), jnp.float32)],
    )(x)


def _mm1_body(x_ref, w_ref, wa_ref, c1_ref, ca_ref,
              hp_ref, a1_ref, a2_ref, gmax_ref):
    nt = pl.program_id(1)
    xb = x_ref[0]
    h = jnp.dot(xb, w_ref[0], preferred_element_type=jnp.float32) + c1_ref[0]
    a = jnp.dot(xb, wa_ref[0], preferred_element_type=jnp.float32) + ca_ref[0]
    hp_ref[0, 0] = h[:, :128]
    hp_ref[0, 1] = h[:, 128:]
    a1_ref[0] = a
    a2_ref[0, 0] = jnp.concatenate([a[:, 0:2], a[:, 4:6]], axis=1)
    a2_ref[0, 1] = jnp.concatenate([a[:, 2:4], a[:, 6:8]], axis=1)
    cm = jnp.max(a, axis=0, keepdims=True)

    @pl.when(nt == 0)
    def _():
        gmax_ref[0] = cm

    @pl.when(nt != 0)
    def _():
        gmax_ref[0] = jnp.maximum(gmax_ref[0], cm)


def _tc_mm1(x, W1e, Wa, c1, ca):
    return pl.pallas_call(
        _mm1_body,
        grid=(B, NT),
        in_specs=[pl.BlockSpec((1, RT, D), lambda b, nt: (b, nt, 0)),
                  pl.BlockSpec((1, D, 256), lambda b, nt: (b, 0, 0)),
                  pl.BlockSpec((1, D, 8), lambda b, nt: (b, 0, 0)),
                  pl.BlockSpec((1, 1, 256), lambda b, nt: (b, 0, 0)),
                  pl.BlockSpec((1, 1, 8), lambda b, nt: (b, 0, 0))],
        out_specs=[pl.BlockSpec((1, 2, RT, 128), lambda b, nt: (b, 0, nt, 0)),
                   pl.BlockSpec((1, RT, 8), lambda b, nt: (b, nt, 0)),
                   pl.BlockSpec((1, 2, RT, 4), lambda b, nt: (b, 0, nt, 0)),
                   pl.BlockSpec((1, 1, 8), lambda b, nt: (b, 0, 0))],
        out_shape=[jax.ShapeDtypeStruct((B, 2, N, 128), jnp.float32),
                   jax.ShapeDtypeStruct((B, N, 8), jnp.float32),
                   jax.ShapeDtypeStruct((B, 2, N, 4), jnp.float32),
                   jax.ShapeDtypeStruct((B, 1, 8), jnp.float32)],
    )(x, W1e, Wa, c1, ca)


def _norm1_body(msg_ref, den_ref, hp_ref, a1_ref, gmax_ref, b1_ref,
                xr_ref, s2_ref, q2_ref):
    nt = pl.program_id(1)
    a = a1_ref[0]
    asrc = a[:, 0:4]
    adst = a[:, 4:8]
    gm = gmax_ref[0, 0, 0:4]
    shift = _leaky(gm[None, :] + adst)
    wself = jnp.exp(_leaky(asrc + adst) - shift)
    den4 = jnp.concatenate([den_ref[0, 0], den_ref[0, 1]], axis=1) + wself
    dinv = 1.0 / (den4 + 1e-16)
    for h in range(4):
        half = h // 2
        col = (h % 2) * 64
        msg = msg_ref[0, half][:, col:col + 64]
        hcols = hp_ref[0, half][:, col:col + 64]
        o = (msg + wself[:, h:h + 1] * hcols) * dinv[:, h:h + 1] \
            + b1_ref[0, 0, h * 64:(h + 1) * 64][None, :]
        xr = jnp.maximum(o, 0.0)
        xr_ref[0, :, h * 64:(h + 1) * 64] = xr
        s = jnp.sum(xr, axis=0, keepdims=True)
        q = jnp.sum(xr * xr, axis=0, keepdims=True)

        @pl.when(nt == 0)
        def _():
            s2_ref[0, :, h * 64:(h + 1) * 64] = s
            q2_ref[0, :, h * 64:(h + 1) * 64] = q

        @pl.when(nt != 0)
        def _():
            s2_ref[0, :, h * 64:(h + 1) * 64] = s2_ref[0, :, h * 64:(h + 1) * 64] + s
            q2_ref[0, :, h * 64:(h + 1) * 64] = q2_ref[0, :, h * 64:(h + 1) * 64] + q


def _tc_norm1(msgacc, den, hp, a1, gmax, bias1):
    return pl.pallas_call(
        _norm1_body,
        grid=(B, NT),
        in_specs=[pl.BlockSpec((1, 2, RT, 128), lambda b, nt: (b, 0, nt, 0)),
                  pl.BlockSpec((1, 2, RT, 2), lambda b, nt: (b, 0, nt, 0)),
                  pl.BlockSpec((1, 2, RT, 128), lambda b, nt: (b, 0, nt, 0)),
                  pl.BlockSpec((1, RT, 8), lambda b, nt: (b, nt, 0)),
                  pl.BlockSpec((1, 1, 8), lambda b, nt: (b, 0, 0)),
                  pl.BlockSpec((1, 1, 256), lambda b, nt: (b, 0, 0))],
        out_specs=[pl.BlockSpec((1, RT, 256), lambda b, nt: (b, nt, 0)),
                   pl.BlockSpec((1, 1, 256), lambda b, nt: (b, 0, 0)),
                   pl.BlockSpec((1, 1, 256), lambda b, nt: (b, 0, 0))],
        out_shape=[jax.ShapeDtypeStruct((B, N, 256), jnp.float32),
                   jax.ShapeDtypeStruct((B, 1, 256), jnp.float32),
                   jax.ShapeDtypeStruct((B, 1, 256), jnp.float32)],
    )(msgacc, den, hp, a1, gmax, bias1)


def _mm2_body(xr_ref, p2_ref, c2_ref, tab_ref, cmax_ref):
    nt = pl.program_id(1)
    t = jnp.dot(xr_ref[0], p2_ref[0], preferred_element_type=jnp.float32) + c2_ref[0]
    tab_ref[0] = t
    cm = jnp.max(t, axis=0, keepdims=True)

    @pl.when(nt == 0)
    def _():
        cmax_ref[0] = cm

    @pl.when(nt != 0)
    def _():
        cmax_ref[0] = jnp.maximum(cmax_ref[0], cm)


def _tc_mm2(xr, P2, c2):
    return pl.pallas_call(
        _mm2_body,
        grid=(B, NT),
        in_specs=[pl.BlockSpec((1, RT, 256), lambda b, nt: (b, nt, 0)),
                  pl.BlockSpec((1, 256, 8), lambda b, nt: (b, 0, 0)),
                  pl.BlockSpec((1, 1, 8), lambda b, nt: (b, 0, 0))],
        out_specs=[pl.BlockSpec((1, RT, 8), lambda b, nt: (b, nt, 0)),
                   pl.BlockSpec((1, 1, 8), lambda b, nt: (b, 0, 0))],
        out_shape=[jax.ShapeDtypeStruct((B, N, 8), jnp.float32),
                   jax.ShapeDtypeStruct((B, 1, 8), jnp.float32)],
    )(xr, P2, c2)


def _norm2_body(acc_ref, tab_ref, gmax_ref, b2_ref, batch_ref,
                ps_ref, pc_ref):
    nt = pl.program_id(1)
    th = tab_ref[0]
    h2 = th[:, 0:4]
    as2 = th[:, 4:5]
    ad2 = th[:, 5:6]
    g2 = gmax_ref[0, 0, 4]
    wself = jnp.exp(_leaky(as2 + ad2) - _leaky(g2 + ad2))
    a0 = acc_ref[0, 0]
    a1_ = acc_ref[1, 0]
    den = a0[:, 4:5] + a1_[:, 4:5] + wself
    msg = a0[:, 0:4] + a1_[:, 0:4] + wself * h2
    o2 = msg / (den + 1e-16) + b2_ref[0, 0][None, :]
    gel = 0.5 * o2 * (1.0 + lax.erf(o2 * 0.7071067811865476))
    oh = (batch_ref[:, 0:1] == lax.broadcasted_iota(jnp.int32, (1, G), 1)
          ).astype(jnp.float32)
    psum = lax.dot_general(oh, gel, (((0,), (0,)), ((), ())),
                           preferred_element_type=jnp.float32)
    ones = jnp.ones((RT, 1), jnp.float32)
    pcnt = lax.dot_general(oh, ones, (((0,), (0,)), ((), ())),
                           preferred_element_type=jnp.float32)

    @pl.when(nt == 0)
    def _():
        ps_ref[0] = psum
        pc_ref[0] = pcnt

    @pl.when(nt != 0)
    def _():
        ps_ref[0] = ps_ref[0] + psum
        pc_ref[0] = pc_ref[0] + pcnt


def _tc_norm2pool(acc2, tab2, gmax2, bias2, batch2d):
    return pl.pallas_call(
        _norm2_body,
        grid=(B, NT),
        in_specs=[pl.BlockSpec((2, 1, RT, 8), lambda b, nt: (0, b, nt, 0)),
                  pl.BlockSpec((1, RT, 8), lambda b, nt: (b, nt, 0)),
                  pl.BlockSpec((1, 1, 8), lambda b, nt: (b, 0, 0)),
                  pl.BlockSpec((1, 1, 4), lambda b, nt: (b, 0, 0)),
                  pl.BlockSpec((RT, 1), lambda b, nt: (nt, 0))],
        out_specs=[pl.BlockSpec((1, G, 4), lambda b, nt: (b, 0, 0)),
                   pl.BlockSpec((1, G, 1), lambda b, nt: (b, 0, 0))],
        out_shape=[jax.ShapeDtypeStruct((B, G, 4), jnp.float32),
                   jax.ShapeDtypeStruct((B, G, 1), jnp.float32)],
    )(acc2, tab2, gmax2, bias2, batch2d)


def _fin_body(ps_ref, pc_ref, wf_ref, bf_ref, out_ref):
    acc = jnp.zeros((G, C), jnp.float32)
    for b in range(B):
        pooled = ps_ref[b] / jnp.maximum(pc_ref[b], 1.0)
        m = jnp.max(pooled, axis=1, keepdims=True)
        z = pooled - m
        ls = z - jnp.log(jnp.sum(jnp.exp(z), axis=1, keepdims=True))
        acc = acc + jnp.dot(ls, wf_ref[4 * b:4 * b + 4, :],
                            preferred_element_type=jnp.float32)
    out_ref[...] = jnp.maximum(acc + bf_ref[0][None, :], 0.0)


def _tc_fin(psum, pcnt, Wf, bf):
    return pl.pallas_call(
        _fin_body,
        out_shape=jax.ShapeDtypeStruct((G, C), jnp.float32),
    )(psum, pcnt, Wf, bf[None, :])


def _sc1_body(a1_h, gmax_h, src_h, dst_h, hp_h,
              msg_h, den_h,
              tab_v, gbuf_v, idxs_v, idxd_v, idxl_v, iw_v, wab_v, wden_v,
              rows_v, zrow_v, zbuf_v, acc_s, dacc_s, sem, sem_s):
    c = lax.axis_index("c")
    sid = lax.axis_index("s")

    def zr(r, _):
        for q in range(8):
            zrow_v[r, pl.ds(q * 16, 16)] = jnp.zeros((16,), jnp.float32)
        return 0

    lax.fori_loop(0, 24, zr, 0)

    def zb(i, _):
        zbuf_v[pl.ds(i * 16, 16)] = jnp.zeros((16,), jnp.float32)
        return 0

    lax.fori_loop(0, 80, zb, 0)

    def chunk_work(b, base, pb, pz):
        pltpu.sync_copy(src_h.at[pl.ds(base, CH1)], idxs_v)
        pltpu.sync_copy(dst_h.at[pl.ds(base, CH1)], idxd_v)
        for j in range(CH1 // 16):
            sv = idxs_v[pl.ds(j * 16, 16)]
            dv = idxd_v[pl.ds(j * 16, 16)]
            inr = (dv >= lo_ref[0]) & (dv < lo_ref[0] + NH1)
            idxl_v[pb, pl.ds(j * 16, 16)] = jnp.where(inr, dv - lo_ref[0], 0)
            for hh in range(2):
                hvec = jnp.full((16,), hh, jnp.int32)
                sa = plsc.load_gather(tab_v, [sv * 4 + hvec])
                da = plsc.load_gather(tab_v, [dv * 4 + hvec + 2])
                gm = plsc.load_gather(gbuf_v, [jnp.broadcast_to(c * 2 + hh, (16,)).astype(jnp.int32)])
                w = jnp.exp(_leaky(sa + da) - _leaky(gm + da))
                wab_v[pl.ds(hh * CH1 + j * 16, 16)] = jnp.where(inr, w, 0.0)
                wden_v[pb, hh, pl.ds(j * 16, 16)] = jnp.where(pz, w, 0.0)
                iw_v[pb, hh, pl.ds(j * 16, 16)] = dv * 2 + hh

        pltpu.async_copy(hp_h.at[b, c].at[idxs_v], rows_v.at[pb], sem).wait()

        def sbody(e, _):
            e16 = jnp.broadcast_to(e, (16,)).astype(jnp.int32)
            wa = plsc.load_gather(wab_v, [e16])
            wb = plsc.load_gather(wab_v, [e16 + CH1])
            for q in range(4):
                rows_v[pb, e, pl.ds(q * 16, 16)] = rows_v[pb, e, pl.ds(q * 16, 16)] * wa
            for q in range(4, 8):
                rows_v[pb, e, pl.ds(q * 16, 16)] = rows_v[pb, e, pl.ds(q * 16, 16)] * wb
            return 0

        lax.fori_loop(0, CH1, sbody, 0)

    def issue_scatter(pb):
        pltpu.async_copy(rows_v.at[pb], acc_s.at[idxl_v.at[pb]], sem_s, add=True)
        pltpu.async_copy(wden_v.at[pb, 0], dacc_s.at[iw_v.at[pb, 0]], sem_s, add=True)
        pltpu.async_copy(wden_v.at[pb, 1], dacc_s.at[iw_v.at[pb, 1]], sem_s, add=True)

    def wait_scatter(pb):
        pltpu.make_async_copy(rows_v.at[pb], acc_s.at[idxl_v.at[pb]], sem_s).wait()
        pltpu.make_async_copy(wden_v.at[pb, 0], dacc_s.at[iw_v.at[pb, 0]], sem_s).wait()
        pltpu.make_async_copy(wden_v.at[pb, 1], dacc_s.at[iw_v.at[pb, 1]], sem_s).wait()

    lo_ref = [None]

    def branch_body(b, _):
        pltpu.sync_copy(zbuf_v, dacc_s.at[pl.ds(sid * 1280, 1280)])
        pltpu.sync_copy(a1_h.at[b, c, 0], tab_v)
        pltpu.sync_copy(gmax_h.at[b, 0], gbuf_v)

        def pass_body(p, _):
            lo_ref[0] = p * NH1
            pz = p == 0
            for i in range(9):
                pltpu.sync_copy(zrow_v,
                                acc_s.at[pl.ds(sid * NH1T + i * 24, 24)])
            plsc.subcore_barrier()

            tile0 = b * E + sid * EPTF
            chunk_work(b, tile0, 0, pz)
            issue_scatter(0)

            def loop_body(k, _):
                pb = lax.rem(k, 2)
                chunk_work(b, tile0 + k * CH1, pb, pz)
                wait_scatter(1 - pb)
                issue_scatter(pb)
                return 0

            lax.fori_loop(1, NCH1, loop_body, 0)
            wait_scatter((NCH1 - 1) % 2)

            @pl.when(sid < 4)
            def _():
                chunk_work(b, b * E + 16 * EPTF + sid * CH1, 0, pz)
                pltpu.sync_copy(rows_v.at[0], acc_s.at[idxl_v.at[0]], add=True)
                pltpu.sync_copy(wden_v.at[0, 0], dacc_s.at[iw_v.at[0, 0]], add=True)
                pltpu.sync_copy(wden_v.at[0, 1], dacc_s.at[iw_v.at[0, 1]], add=True)

            plsc.subcore_barrier()
            pltpu.sync_copy(acc_s.at[pl.ds(sid * NH1T, NH1T)],
                            msg_h.at[b, c, p, pl.ds(sid * NH1T, NH1T)])
            plsc.subcore_barrier()
            return 0

        lax.fori_loop(0, 3, pass_body, 0)

        @pl.when(sid == 0)
        def _():
            pltpu.sync_copy(dacc_s, den_h.at[b, c, 0])

        plsc.subcore_barrier()
        return 0

    lax.fori_loop(0, B, branch_body, 0)


def _sc_layer1(a1, gmax16, src_flat, dst_flat, hp):
    mesh = plsc.VectorSubcoreMesh(core_axis_name="c", subcore_axis_name="s")
    f = pl.kernel(
        _sc1_body,
        mesh=mesh,
        compiler_params=pltpu.CompilerParams(needs_layout_passes=False),
        out_type=[jax.ShapeDtypeStruct((B, 2, 3, NH1, 128), jnp.float32),
                  jax.ShapeDtypeStruct((B, 2, 1, 20480), jnp.float32)],
        scratch_types=[
            pltpu.VMEM((N * 4,), jnp.float32),
            pltpu.VMEM((16,), jnp.float32),
            pltpu.VMEM((CH1,), jnp.int32),
            pltpu.VMEM((CH1,), jnp.int32),
            pltpu.VMEM((2, CH1), jnp.int32),
            pltpu.VMEM((2, 2, CH1), jnp.int32),
            pltpu.VMEM((2 * CH1,), jnp.float32),
            pltpu.VMEM((2, 2, CH1), jnp.float32),
            pltpu.VMEM((2, CH1, 128), jnp.float32),
            pltpu.VMEM((24, 128), jnp.float32),
            pltpu.VMEM((1280,), jnp.float32),
            pltpu.VMEM_SHARED((NH1, 128), jnp.float32),
            pltpu.VMEM_SHARED((20480,), jnp.float32),
            pltpu.SemaphoreType.DMA,
            pltpu.SemaphoreType.DMA,
        ],
    )
    return f(a1, gmax16, src_flat, dst_flat, hp)


def _sc2_body(tab_h, gmax_h, src_h, dst_h,
              acc_h,
              tab_v, gbuf_v, idxs_v, idxd_v, i8_v, val_v, zbuf_v, acc8_s):
    c = lax.axis_index("c")
    sid = lax.axis_index("s")
    four16 = jnp.full((16,), 4, jnp.int32)
    five16 = jnp.full((16,), 5, jnp.int32)

    def zb(i, _):
        zbuf_v[pl.ds(i * 16, 16)] = jnp.zeros((16,), jnp.float32)
        return 0

    lax.fori_loop(0, 80, zb, 0)

    for b in range(B):
        pltpu.sync_copy(tab_h.at[b, 0], tab_v)
        pltpu.sync_copy(gmax_h.at[b, 0], gbuf_v)
        for p in range(2):
            lo = p * NH
            for i in range(2):
                pltpu.sync_copy(zbuf_v,
                                acc8_s.at[pl.ds(sid * 2560 + i * 1280, 1280)])
            plsc.subcore_barrier()

            def chunk(k, _):
                base = b * E + (sid * 2 + c) * EPT32 + k * CH
                pltpu.sync_copy(src_h.at[pl.ds(base, CH)], idxs_v)
                pltpu.sync_copy(dst_h.at[pl.ds(base, CH)], idxd_v)
                for j in range(CH // 16):
                    sv = idxs_v[pl.ds(j * 16, 16)]
                    dv = idxd_v[pl.ds(j * 16, 16)]
                    inr = (dv >= lo) & (dv < lo + NH)
                    dvl = jnp.where(inr, dv - lo, 0)
                    as2 = plsc.load_gather(tab_v, [sv * 8 + four16])
                    ad2 = plsc.load_gather(tab_v, [dv * 8 + five16])
                    g2 = plsc.load_gather(gbuf_v, [four16])
                    w = jnp.exp(_leaky(as2 + ad2) - _leaky(g2 + ad2))
                    w = jnp.where(inr, w, 0.0)
                    val_v[4, pl.ds(j * 16, 16)] = w
                    i8_v[4, pl.ds(j * 16, 16)] = dvl * 8 + 4
                    for cc in range(4):
                        cc16 = jnp.full((16,), cc, jnp.int32)
                        hcc = plsc.load_gather(tab_v, [sv * 8 + cc16])
                        val_v[cc, pl.ds(j * 16, 16)] = w * hcc
                        i8_v[cc, pl.ds(j * 16, 16)] = dvl * 8 + cc
                for cc in range(5):
                    pltpu.sync_copy(val_v.at[cc], acc8_s.at[i8_v.at[cc]], add=True)
                return 0

            lax.fori_loop(0, EPT32 // CH, chunk, 0)
            plsc.subcore_barrier()

            @pl.when(sid == 0)
            def _():
                pltpu.sync_copy(acc8_s, acc_h.at[b, c, p])

            plsc.subcore_barrier()


def _sc_layer2(tab2, gmax16, src_flat, dst_flat):
    mesh = plsc.VectorSubcoreMesh(core_axis_name="c", subcore_axis_name="s")
    f = pl.kernel(
        _sc2_body,
        mesh=mesh,
        compiler_params=pltpu.CompilerParams(needs_layout_passes=False),
        out_type=[jax.ShapeDtypeStruct((B, 2, 2, 8 * NH), jnp.float32)],
        scratch_types=[
            pltpu.VMEM((N * 8,), jnp.float32),
            pltpu.VMEM((16,), jnp.float32),
            pltpu.VMEM((CH,), jnp.int32),
            pltpu.VMEM((CH,), jnp.int32),
            pltpu.VMEM((5, CH), jnp.int32),
            pltpu.VMEM((5, CH), jnp.float32),
            pltpu.VMEM((1280,), jnp.float32),
            pltpu.VMEM_SHARED((8 * NH,), jnp.float32),
        ],
    )
    return f(tab2, gmax16, src_flat, dst_flat)[0]


def kernel(x, edge_index, batch, gamma1, beta1, W1, att_src1, att_dst1, bias1,
           gamma2, beta2, W2, att_src2, att_dst2, bias2, Wf, bf):
    ei = edge_index.astype(jnp.int32)

    s0, q0 = _tc_stats(x, D)
    mu = s0[:, 0, :] / N
    var = q0[:, 0, :] / N - mu * mu
    sc1 = gamma1 / jnp.sqrt(var + 1e-5)
    sh1 = beta1 - mu * sc1
    W1e = W1 * sc1[:, :, None]
    c1 = jnp.einsum("bd,bdk->bk", sh1, W1)[:, None, :]
    m4 = (jnp.arange(256)[:, None] // 64 == jnp.arange(4)[None, :]).astype(jnp.float32)
    A8 = jnp.concatenate([att_src1.reshape(B, 256, 1) * m4[None],
                          att_dst1.reshape(B, 256, 1) * m4[None]], axis=2)
    Wa = jnp.einsum("bdk,bkh->bdh", W1e, A8)
    ca = jnp.einsum("bk,bkh->bh", c1[:, 0, :], A8)[:, None, :]

    hp, a1, a2, gmax1 = _tc_mm1(x, W1e, Wa, c1, ca)
    g16 = jnp.pad(gmax1, ((0, 0), (0, 0), (0, 8)))
    src_flat = ei[:, 0, :].reshape(B * E)
    dst_flat = ei[:, 1, :].reshape(B * E)

    a2f = a2.reshape(B, 2, 1, N * 4)
    msgacc, denflat = _sc_layer1(a2f, g16, src_flat, dst_flat, hp)
    msgacc = msgacc.reshape(B, 2, NP1, 128)
    den = denflat[:, :, 0, :2 * N].reshape(B, 2, N, 2)

    xr, s2, q2 = _tc_norm1(msgacc, den, hp, a1, gmax1, bias1[:, None, :])
    mu2 = s2[:, 0, :] / N
    var2 = q2[:, 0, :] / N - mu2 * mu2
    sc2 = gamma2 / jnp.sqrt(var2 + 1e-5)
    sh2 = beta2 - mu2 * sc2
    W2e = W2 * sc2[:, :, None]
    c2 = jnp.einsum("bd,bdk->bk", sh2, W2)
    as2v = att_src2[:, 0, :]
    ad2v = att_dst2[:, 0, :]
    P2 = jnp.concatenate([
        W2e,
        jnp.einsum("bdk,bk->bd", W2e, as2v)[:, :, None],
        jnp.einsum("bdk,bk->bd", W2e, ad2v)[:, :, None],
        jnp.zeros((B, 256, 2), jnp.float32),
    ], axis=2)
    c2cat = jnp.concatenate([
        c2,
        jnp.einsum("bk,bk->b", c2, as2v)[:, None],
        jnp.einsum("bk,bk->b", c2, ad2v)[:, None],
        jnp.zeros((B, 2), jnp.float32),
    ], axis=1)[:, None, :]

    tab2, cmax2 = _tc_mm2(xr, P2, c2cat)
    g216 = jnp.pad(cmax2, ((0, 0), (0, 0), (0, 8)))

    tab2f = tab2.reshape(B, 1, N * 8)
    acc2 = _sc_layer2(tab2f, g216, src_flat, dst_flat).reshape(B, 2, NP, 8)
    acc2 = jnp.transpose(acc2, (1, 0, 2, 3))

    psum, pcnt = _tc_norm2pool(acc2, tab2, cmax2, bias2[:, None, :],
                               batch.astype(jnp.int32)[:, None])

    return _tc_fin(psum, pcnt, Wf, bf)


# speedup vs baseline: 29.0173x; 1.3418x over previous
import functools

import jax
import jax.numpy as jnp
from jax import lax
from jax.experimental import pallas as pl
from jax.experimental.pallas import tpu as pltpu
from jax.experimental.pallas import tpu_sc as plsc

B = 4
N = 10000
E = 320000
D = 128
H = 4
F = 64
C = 4
G = 64

RT = 1000
NT = N // RT
CH = 80
CH1 = 128
EPTF = 19968
NCH1 = EPTF // CH1
EPT16 = E // 16
EPT32 = E // 32
NP = 10240
NPT = NP // 16
NH1 = 3456
NP1 = 3 * NH1
NH1T = NH1 // 16
NH = 5120
NHT = NH // 16


def _leaky(x):
    return jnp.where(x >= 0, x, 0.2 * x)


def _stats_body(x_ref, s_ref, q_ref):
    nt = pl.program_id(1)
    xb = x_ref[0]
    s = jnp.sum(xb, axis=0, keepdims=True)
    q = jnp.sum(xb * xb, axis=0, keepdims=True)

    @pl.when(nt == 0)
    def _():
        s_ref[0] = s
        q_ref[0] = q

    @pl.when(nt != 0)
    def _():
        s_ref[0] = s_ref[0] + s
        q_ref[0] = q_ref[0] + q


def _tc_stats(x, d):
    return pl.pallas_call(
        _stats_body,
        grid=(B, NT),
        in_specs=[pl.BlockSpec((1, RT, d), lambda b, nt: (b, nt, 0))],
        out_specs=[pl.BlockSpec((1, 1, d), lambda b, nt: (b, 0, 0)),
                   pl.BlockSpec((1, 1, d), lambda b, nt: (b, 0, 0))],
        out_shape=[jax.ShapeDtypeStruct((B, 1, d), jnp.float32),
                   jax.ShapeDtypeStruct((B, 1, d), jnp.float32)],
    )(x)


def _mm1_body(x_ref, w_ref, wa_ref, c1_ref, ca_ref,
              hp_ref, a1_ref, a2_ref, gmax_ref):
    nt = pl.program_id(1)
    xb = x_ref[0]
    h = jnp.dot(xb, w_ref[0], preferred_element_type=jnp.float32) + c1_ref[0]
    a = jnp.dot(xb, wa_ref[0], preferred_element_type=jnp.float32) + ca_ref[0]
    hp_ref[0, 0] = h[:, :128]
    hp_ref[0, 1] = h[:, 128:]
    a1_ref[0] = a
    a2_ref[0, 0] = jnp.concatenate([a[:, 0:2], a[:, 4:6]], axis=1)
    a2_ref[0, 1] = jnp.concatenate([a[:, 2:4], a[:, 6:8]], axis=1)
    cm = jnp.max(a, axis=0, keepdims=True)

    @pl.when(nt == 0)
    def _():
        gmax_ref[0] = cm

    @pl.when(nt != 0)
    def _():
        gmax_ref[0] = jnp.maximum(gmax_ref[0], cm)


def _tc_mm1(x, W1e, Wa, c1, ca):
    return pl.pallas_call(
        _mm1_body,
        grid=(B, NT),
        in_specs=[pl.BlockSpec((1, RT, D), lambda b, nt: (b, nt, 0)),
                  pl.BlockSpec((1, D, 256), lambda b, nt: (b, 0, 0)),
                  pl.BlockSpec((1, D, 8), lambda b, nt: (b, 0, 0)),
                  pl.BlockSpec((1, 1, 256), lambda b, nt: (b, 0, 0)),
                  pl.BlockSpec((1, 1, 8), lambda b, nt: (b, 0, 0))],
        out_specs=[pl.BlockSpec((1, 2, RT, 128), lambda b, nt: (b, 0, nt, 0)),
                   pl.BlockSpec((1, RT, 8), lambda b, nt: (b, nt, 0)),
                   pl.BlockSpec((1, 2, RT, 4), lambda b, nt: (b, 0, nt, 0)),
                   pl.BlockSpec((1, 1, 8), lambda b, nt: (b, 0, 0))],
        out_shape=[jax.ShapeDtypeStruct((B, 2, N, 128), jnp.float32),
                   jax.ShapeDtypeStruct((B, N, 8), jnp.float32),
                   jax.ShapeDtypeStruct((B, 2, N, 4), jnp.float32),
                   jax.ShapeDtypeStruct((B, 1, 8), jnp.float32)],
    )(x, W1e, Wa, c1, ca)


def _norm1_body(msg_ref, den_ref, hp_ref, a1_ref, gmax_ref, b1_ref,
                xr_ref, s2_ref, q2_ref):
    nt = pl.program_id(1)
    a = a1_ref[0]
    asrc = a[:, 0:4]
    adst = a[:, 4:8]
    gm = gmax_ref[0, 0, 0:4]
    shift = _leaky(gm[None, :] + adst)
    wself = jnp.exp(_leaky(asrc + adst) - shift)
    den4 = jnp.concatenate([den_ref[0, 0], den_ref[0, 1]], axis=1) + wself
    dinv = 1.0 / (den4 + 1e-16)
    for h in range(4):
        half = h // 2
        col = (h % 2) * 64
        msg = msg_ref[0, half][:, col:col + 64]
        hcols = hp_ref[0, half][:, col:col + 64]
        o = (msg + wself[:, h:h + 1] * hcols) * dinv[:, h:h + 1] \
            + b1_ref[0, 0, h * 64:(h + 1) * 64][None, :]
        xr = jnp.maximum(o, 0.0)
        xr_ref[0, :, h * 64:(h + 1) * 64] = xr
        s = jnp.sum(xr, axis=0, keepdims=True)
        q = jnp.sum(xr * xr, axis=0, keepdims=True)

        @pl.when(nt == 0)
        def _():
            s2_ref[0, :, h * 64:(h + 1) * 64] = s
            q2_ref[0, :, h * 64:(h + 1) * 64] = q

        @pl.when(nt != 0)
        def _():
            s2_ref[0, :, h * 64:(h + 1) * 64] = s2_ref[0, :, h * 64:(h + 1) * 64] + s
            q2_ref[0, :, h * 64:(h + 1) * 64] = q2_ref[0, :, h * 64:(h + 1) * 64] + q


def _tc_norm1(msgacc, den, hp, a1, gmax, bias1):
    return pl.pallas_call(
        _norm1_body,
        grid=(B, NT),
        in_specs=[pl.BlockSpec((1, 2, RT, 128), lambda b, nt: (b, 0, nt, 0)),
                  pl.BlockSpec((1, 2, RT, 2), lambda b, nt: (b, 0, nt, 0)),
                  pl.BlockSpec((1, 2, RT, 128), lambda b, nt: (b, 0, nt, 0)),
                  pl.BlockSpec((1, RT, 8), lambda b, nt: (b, nt, 0)),
                  pl.BlockSpec((1, 1, 8), lambda b, nt: (b, 0, 0)),
                  pl.BlockSpec((1, 1, 256), lambda b, nt: (b, 0, 0))],
        out_specs=[pl.BlockSpec((1, RT, 256), lambda b, nt: (b, nt, 0)),
                   pl.BlockSpec((1, 1, 256), lambda b, nt: (b, 0, 0)),
                   pl.BlockSpec((1, 1, 256), lambda b, nt: (b, 0, 0))],
        out_shape=[jax.ShapeDtypeStruct((B, N, 256), jnp.float32),
                   jax.ShapeDtypeStruct((B, 1, 256), jnp.float32),
                   jax.ShapeDtypeStruct((B, 1, 256), jnp.float32)],
    )(msgacc, den, hp, a1, gmax, bias1)


def _mm2_body(xr_ref, p2_ref, c2_ref, tab_ref, cmax_ref):
    nt = pl.program_id(1)
    t = jnp.dot(xr_ref[0], p2_ref[0], preferred_element_type=jnp.float32) + c2_ref[0]
    tab_ref[0] = t
    cm = jnp.max(t, axis=0, keepdims=True)

    @pl.when(nt == 0)
    def _():
        cmax_ref[0] = cm

    @pl.when(nt != 0)
    def _():
        cmax_ref[0] = jnp.maximum(cmax_ref[0], cm)


def _tc_mm2(xr, P2, c2):
    return pl.pallas_call(
        _mm2_body,
        grid=(B, NT),
        in_specs=[pl.BlockSpec((1, RT, 256), lambda b, nt: (b, nt, 0)),
                  pl.BlockSpec((1, 256, 8), lambda b, nt: (b, 0, 0)),
                  pl.BlockSpec((1, 1, 8), lambda b, nt: (b, 0, 0))],
        out_specs=[pl.BlockSpec((1, RT, 8), lambda b, nt: (b, nt, 0)),
                   pl.BlockSpec((1, 1, 8), lambda b, nt: (b, 0, 0))],
        out_shape=[jax.ShapeDtypeStruct((B, N, 8), jnp.float32),
                   jax.ShapeDtypeStruct((B, 1, 8), jnp.float32)],
    )(xr, P2, c2)


def _norm2_body(acc_ref, tab_ref, gmax_ref, b2_ref, batch_ref,
                ps_ref, pc_ref):
    nt = pl.program_id(1)
    th = tab_ref[0]
    h2 = th[:, 0:4]
    as2 = th[:, 4:5]
    ad2 = th[:, 5:6]
    g2 = gmax_ref[0, 0, 4]
    wself = jnp.exp(_leaky(as2 + ad2) - _leaky(g2 + ad2))
    a0 = acc_ref[0, 0]
    a1_ = acc_ref[1, 0]
    den = a0[:, 4:5] + a1_[:, 4:5] + wself
    msg = a0[:, 0:4] + a1_[:, 0:4] + wself * h2
    o2 = msg / (den + 1e-16) + b2_ref[0, 0][None, :]
    gel = 0.5 * o2 * (1.0 + lax.erf(o2 * 0.7071067811865476))
    oh = (batch_ref[:, 0:1] == lax.broadcasted_iota(jnp.int32, (1, G), 1)
          ).astype(jnp.float32)
    psum = lax.dot_general(oh, gel, (((0,), (0,)), ((), ())),
                           preferred_element_type=jnp.float32)
    ones = jnp.ones((RT, 1), jnp.float32)
    pcnt = lax.dot_general(oh, ones, (((0,), (0,)), ((), ())),
                           preferred_element_type=jnp.float32)

    @pl.when(nt == 0)
    def _():
        ps_ref[0] = psum
        pc_ref[0] = pcnt

    @pl.when(nt != 0)
    def _():
        ps_ref[0] = ps_ref[0] + psum
        pc_ref[0] = pc_ref[0] + pcnt


def _tc_norm2pool(acc2, tab2, gmax2, bias2, batch2d):
    return pl.pallas_call(
        _norm2_body,
        grid=(B, NT),
        in_specs=[pl.BlockSpec((2, 1, RT, 8), lambda b, nt: (0, b, nt, 0)),
                  pl.BlockSpec((1, RT, 8), lambda b, nt: (b, nt, 0)),
                  pl.BlockSpec((1, 1, 8), lambda b, nt: (b, 0, 0)),
                  pl.BlockSpec((1, 1, 4), lambda b, nt: (b, 0, 0)),
                  pl.BlockSpec((RT, 1), lambda b, nt: (nt, 0))],
        out_specs=[pl.BlockSpec((1, G, 4), lambda b, nt: (b, 0, 0)),
                   pl.BlockSpec((1, G, 1), lambda b, nt: (b, 0, 0))],
        out_shape=[jax.ShapeDtypeStruct((B, G, 4), jnp.float32),
                   jax.ShapeDtypeStruct((B, G, 1), jnp.float32)],
    )(acc2, tab2, gmax2, bias2, batch2d)


def _fin_body(ps_ref, pc_ref, wf_ref, bf_ref, out_ref):
    acc = jnp.zeros((G, C), jnp.float32)
    for b in range(B):
        pooled = ps_ref[b] / jnp.maximum(pc_ref[b], 1.0)
        m = jnp.max(pooled, axis=1, keepdims=True)
        z = pooled - m
        ls = z - jnp.log(jnp.sum(jnp.exp(z), axis=1, keepdims=True))
        acc = acc + jnp.dot(ls, wf_ref[4 * b:4 * b + 4, :],
                            preferred_element_type=jnp.float32)
    out_ref[...] = jnp.maximum(acc + bf_ref[0][None, :], 0.0)


def _tc_fin(psum, pcnt, Wf, bf):
    return pl.pallas_call(
        _fin_body,
        out_shape=jax.ShapeDtypeStruct((G, C), jnp.float32),
    )(psum, pcnt, Wf, bf[None, :])


def _sc1_body(a1_h, gmax_h, src_h, dst_h, hp_h,
              msg_h, den_h,
              tab_v, gbuf_v, idxs_v, idxd_v, idxl_v, iw_v, wab_v, wden_v,
              rows_v, zrow_v, zbuf_v, acc_s, dacc_s, sem, sem_s):
    c = lax.axis_index("c")
    sid = lax.axis_index("s")

    def zr(r, _):
        for q in range(8):
            zrow_v[r, pl.ds(q * 16, 16)] = jnp.zeros((16,), jnp.float32)
        return 0

    lax.fori_loop(0, 24, zr, 0)

    def zb(i, _):
        zbuf_v[pl.ds(i * 16, 16)] = jnp.zeros((16,), jnp.float32)
        return 0

    lax.fori_loop(0, 80, zb, 0)

    lo_ref = [None]

    def load_idx(base, ib):
        pltpu.sync_copy(src_h.at[pl.ds(base, CH1)], idxs_v.at[ib])
        pltpu.sync_copy(dst_h.at[pl.ds(base, CH1)], idxd_v.at[ib])

    def w_phase(pb, pz):
        lo = lo_ref[0]
        for j in range(CH1 // 16):
            sv = idxs_v[pb, pl.ds(j * 16, 16)]
            dv = idxd_v[pb, pl.ds(j * 16, 16)]
            inr = (dv >= lo) & (dv < lo + NH1)
            idxl_v[pb, pl.ds(j * 16, 16)] = jnp.where(inr, dv - lo, 0)
            for hh in range(2):
                hvec = jnp.full((16,), hh, jnp.int32)
                sa = plsc.load_gather(tab_v, [sv * 4 + hvec])
                da = plsc.load_gather(tab_v, [dv * 4 + hvec + 2])
                gm = plsc.load_gather(
                    gbuf_v, [jnp.broadcast_to(c * 2 + hh, (16,)).astype(jnp.int32)])
                w = jnp.exp(_leaky(sa + da) - _leaky(gm + da))
                wab_v[pl.ds(hh * CH1 + j * 16, 16)] = jnp.where(inr, w, 0.0)
                wden_v[pb, hh, pl.ds(j * 16, 16)] = jnp.where(pz, w, 0.0)
                iw_v[pb, hh, pl.ds(j * 16, 16)] = dv * 2 + hh

    def issue_gather(b, pb):
        pltpu.async_copy(hp_h.at[b, c].at[idxs_v.at[pb]], rows_v.at[pb], sem)

    def wait_gather(b, pb):
        pltpu.make_async_copy(hp_h.at[b, c].at[idxs_v.at[pb]],
                              rows_v.at[pb], sem).wait()

    def scale(pb):
        def sbody(e2, _):
            for u in range(2):
                e = e2 * 2 + u
                e16 = jnp.broadcast_to(e, (16,)).astype(jnp.int32)
                wa = plsc.load_gather(wab_v, [e16])
                wb = plsc.load_gather(wab_v, [e16 + CH1])
                for q in range(4):
                    rows_v[pb, e, pl.ds(q * 16, 16)] = \
                        rows_v[pb, e, pl.ds(q * 16, 16)] * wa
                for q in range(4, 8):
                    rows_v[pb, e, pl.ds(q * 16, 16)] = \
                        rows_v[pb, e, pl.ds(q * 16, 16)] * wb
            return 0

        lax.fori_loop(0, CH1 // 2, sbody, 0)

    def issue_scatter(pb):
        pltpu.async_copy(rows_v.at[pb], acc_s.at[idxl_v.at[pb]], sem_s, add=True)
        pltpu.async_copy(wden_v.at[pb, 0], dacc_s.at[iw_v.at[pb, 0]], sem_s, add=True)
        pltpu.async_copy(wden_v.at[pb, 1], dacc_s.at[iw_v.at[pb, 1]], sem_s, add=True)

    def wait_scatter(pb):
        pltpu.make_async_copy(rows_v.at[pb], acc_s.at[idxl_v.at[pb]], sem_s).wait()
        pltpu.make_async_copy(wden_v.at[pb, 0], dacc_s.at[iw_v.at[pb, 0]], sem_s).wait()
        pltpu.make_async_copy(wden_v.at[pb, 1], dacc_s.at[iw_v.at[pb, 1]], sem_s).wait()

    def branch_body(b, _):
        pltpu.sync_copy(zbuf_v, dacc_s.at[pl.ds(sid * 1280, 1280)])
        pltpu.sync_copy(a1_h.at[b, c, 0], tab_v)
        pltpu.sync_copy(gmax_h.at[b, 0], gbuf_v)

        def pass_body(p, _):
            lo_ref[0] = p * NH1
            pz = p == 0
            for i in range(9):
                pltpu.sync_copy(zrow_v,
                                acc_s.at[pl.ds(sid * NH1T + i * 24, 24)])
            plsc.subcore_barrier()

            tile0 = b * E + sid * EPTF
            load_idx(tile0, 0)
            issue_gather(b, 0)
            load_idx(tile0 + CH1, 1)
            w_phase(0, pz)
            issue_gather(b, 1)
            wait_gather(b, 0)
            scale(0)
            issue_scatter(0)

            def loop_body(k, _):
                pb = lax.rem(k, 2)
                load_idx(tile0 + (k + 1) * CH1, 1 - pb)
                w_phase(pb, pz)
                wait_scatter(1 - pb)
                issue_gather(b, 1 - pb)
                wait_gather(b, pb)
                scale(pb)
                issue_scatter(pb)
                return 0

            lax.fori_loop(1, NCH1, loop_body, 0)
            wait_scatter((NCH1 - 1) % 2)
            wait_gather(b, NCH1 % 2)

            @pl.when(sid < 4)
            def _():
                load_idx(b * E + 16 * EPTF + sid * CH1, 0)
                w_phase(0, pz)
                pltpu.async_copy(hp_h.at[b, c].at[idxs_v.at[0]],
                                 rows_v.at[0], sem).wait()
                scale(0)
                pltpu.sync_copy(rows_v.at[0], acc_s.at[idxl_v.at[0]], add=True)
                pltpu.sync_copy(wden_v.at[0, 0], dacc_s.at[iw_v.at[0, 0]], add=True)
                pltpu.sync_copy(wden_v.at[0, 1], dacc_s.at[iw_v.at[0, 1]], add=True)

            plsc.subcore_barrier()
            pltpu.sync_copy(acc_s.at[pl.ds(sid * NH1T, NH1T)],
                            msg_h.at[b, c, p, pl.ds(sid * NH1T, NH1T)])
            plsc.subcore_barrier()
            return 0

        lax.fori_loop(0, 3, pass_body, 0)

        @pl.when(sid == 0)
        def _():
            pltpu.sync_copy(dacc_s, den_h.at[b, c, 0])

        plsc.subcore_barrier()
        return 0

    lax.fori_loop(0, B, branch_body, 0)


def _sc_layer1(a1, gmax16, src_flat, dst_flat, hp):
    mesh = plsc.VectorSubcoreMesh(core_axis_name="c", subcore_axis_name="s")
    f = pl.kernel(
        _sc1_body,
        mesh=mesh,
        compiler_params=pltpu.CompilerParams(needs_layout_passes=False),
        out_type=[jax.ShapeDtypeStruct((B, 2, 3, NH1, 128), jnp.float32),
                  jax.ShapeDtypeStruct((B, 2, 1, 20480), jnp.float32)],
        scratch_types=[
            pltpu.VMEM((N * 4,), jnp.float32),
            pltpu.VMEM((16,), jnp.float32),
            pltpu.VMEM((2, CH1), jnp.int32),
            pltpu.VMEM((2, CH1), jnp.int32),
            pltpu.VMEM((2, CH1), jnp.int32),
            pltpu.VMEM((2, 2, CH1), jnp.int32),
            pltpu.VMEM((2 * CH1,), jnp.float32),
            pltpu.VMEM((2, 2, CH1), jnp.float32),
            pltpu.VMEM((2, CH1, 128), jnp.float32),
            pltpu.VMEM((24, 128), jnp.float32),
            pltpu.VMEM((1280,), jnp.float32),
            pltpu.VMEM_SHARED((NH1, 128), jnp.float32),
            pltpu.VMEM_SHARED((20480,), jnp.float32),
            pltpu.SemaphoreType.DMA,
            pltpu.SemaphoreType.DMA,
        ],
    )
    return f(a1, gmax16, src_flat, dst_flat, hp)


def _sc2_body(tab_h, gmax_h, src_h, dst_h,
              acc_h,
              tab_v, gbuf_v, idxs_v, idxd_v, i8_v, val_v, zbuf_v, acc8_s):
    c = lax.axis_index("c")
    sid = lax.axis_index("s")
    four16 = jnp.full((16,), 4, jnp.int32)
    five16 = jnp.full((16,), 5, jnp.int32)

    def zb(i, _):
        zbuf_v[pl.ds(i * 16, 16)] = jnp.zeros((16,), jnp.float32)
        return 0

    lax.fori_loop(0, 80, zb, 0)

    for b in range(B):
        pltpu.sync_copy(tab_h.at[b, 0], tab_v)
        pltpu.sync_copy(gmax_h.at[b, 0], gbuf_v)
        for p in range(2):
            lo = p * NH
            for i in range(2):
                pltpu.sync_copy(zbuf_v,
                                acc8_s.at[pl.ds(sid * 2560 + i * 1280, 1280)])
            plsc.subcore_barrier()

            def chunk(k, _):
                base = b * E + (sid * 2 + c) * EPT32 + k * CH
                pltpu.sync_copy(src_h.at[pl.ds(base, CH)], idxs_v)
                pltpu.sync_copy(dst_h.at[pl.ds(base, CH)], idxd_v)
                for j in range(CH // 16):
                    sv = idxs_v[pl.ds(j * 16, 16)]
                    dv = idxd_v[pl.ds(j * 16, 16)]
                    inr = (dv >= lo) & (dv < lo + NH)
                    dvl = jnp.where(inr, dv - lo, 0)
                    as2 = plsc.load_gather(tab_v, [sv * 8 + four16])
                    ad2 = plsc.load_gather(tab_v, [dv * 8 + five16])
                    g2 = plsc.load_gather(gbuf_v, [four16])
                    w = jnp.exp(_leaky(as2 + ad2) - _leaky(g2 + ad2))
                    w = jnp.where(inr, w, 0.0)
                    val_v[4, pl.ds(j * 16, 16)] = w
                    i8_v[4, pl.ds(j * 16, 16)] = dvl * 8 + 4
                    for cc in range(4):
                        cc16 = jnp.full((16,), cc, jnp.int32)
                        hcc = plsc.load_gather(tab_v, [sv * 8 + cc16])
                        val_v[cc, pl.ds(j * 16, 16)] = w * hcc
                        i8_v[cc, pl.ds(j * 16, 16)] = dvl * 8 + cc
                for cc in range(5):
                    pltpu.sync_copy(val_v.at[cc], acc8_s.at[i8_v.at[cc]], add=True)
                return 0

            lax.fori_loop(0, EPT32 // CH, chunk, 0)
            plsc.subcore_barrier()

            @pl.when(sid == 0)
            def _():
                pltpu.sync_copy(acc8_s, acc_h.at[b, c, p])

            plsc.subcore_barrier()


def _sc_layer2(tab2, gmax16, src_flat, dst_flat):
    mesh = plsc.VectorSubcoreMesh(core_axis_name="c", subcore_axis_name="s")
    f = pl.kernel(
        _sc2_body,
        mesh=mesh,
        compiler_params=pltpu.CompilerParams(needs_layout_passes=False),
        out_type=[jax.ShapeDtypeStruct((B, 2, 2, 8 * NH), jnp.float32)],
        scratch_types=[
            pltpu.VMEM((N * 8,), jnp.float32),
            pltpu.VMEM((16,), jnp.float32),
            pltpu.VMEM((CH,), jnp.int32),
            pltpu.VMEM((CH,), jnp.int32),
            pltpu.VMEM((5, CH), jnp.int32),
            pltpu.VMEM((5, CH), jnp.float32),
            pltpu.VMEM((1280,), jnp.float32),
            pltpu.VMEM_SHARED((8 * NH,), jnp.float32),
        ],
    )
    return f(tab2, gmax16, src_flat, dst_flat)[0]


def kernel(x, edge_index, batch, gamma1, beta1, W1, att_src1, att_dst1, bias1,
           gamma2, beta2, W2, att_src2, att_dst2, bias2, Wf, bf):
    ei = edge_index.astype(jnp.int32)

    s0, q0 = _tc_stats(x, D)
    mu = s0[:, 0, :] / N
    var = q0[:, 0, :] / N - mu * mu
    sc1 = gamma1 / jnp.sqrt(var + 1e-5)
    sh1 = beta1 - mu * sc1
    W1e = W1 * sc1[:, :, None]
    c1 = jnp.einsum("bd,bdk->bk", sh1, W1)[:, None, :]
    m4 = (jnp.arange(256)[:, None] // 64 == jnp.arange(4)[None, :]).astype(jnp.float32)
    A8 = jnp.concatenate([att_src1.reshape(B, 256, 1) * m4[None],
                          att_dst1.reshape(B, 256, 1) * m4[None]], axis=2)
    Wa = jnp.einsum("bdk,bkh->bdh", W1e, A8)
    ca = jnp.einsum("bk,bkh->bh", c1[:, 0, :], A8)[:, None, :]

    hp, a1, a2, gmax1 = _tc_mm1(x, W1e, Wa, c1, ca)
    g16 = jnp.pad(gmax1, ((0, 0), (0, 0), (0, 8)))
    src_flat = ei[:, 0, :].reshape(B * E)
    dst_flat = ei[:, 1, :].reshape(B * E)

    a2f = a2.reshape(B, 2, 1, N * 4)
    msgacc, denflat = _sc_layer1(a2f, g16, src_flat, dst_flat, hp)
    msgacc = msgacc.reshape(B, 2, NP1, 128)
    den = denflat[:, :, 0, :2 * N].reshape(B, 2, N, 2)

    xr, s2, q2 = _tc_norm1(msgacc, den, hp, a1, gmax1, bias1[:, None, :])
    mu2 = s2[:, 0, :] / N
    var2 = q2[:, 0, :] / N - mu2 * mu2
    sc2 = gamma2 / jnp.sqrt(var2 + 1e-5)
    sh2 = beta2 - mu2 * sc2
    W2e = W2 * sc2[:, :, None]
    c2 = jnp.einsum("bd,bdk->bk", sh2, W2)
    as2v = att_src2[:, 0, :]
    ad2v = att_dst2[:, 0, :]
    P2 = jnp.concatenate([
        W2e,
        jnp.einsum("bdk,bk->bd", W2e, as2v)[:, :, None],
        jnp.einsum("bdk,bk->bd", W2e, ad2v)[:, :, None],
        jnp.zeros((B, 256, 2), jnp.float32),
    ], axis=2)
    c2cat = jnp.concatenate([
        c2,
        jnp.einsum("bk,bk->b", c2, as2v)[:, None],
        jnp.einsum("bk,bk->b", c2, ad2v)[:, None],
        jnp.zeros((B, 2), jnp.float32),
    ], axis=1)[:, None, :]

    tab2, cmax2 = _tc_mm2(xr, P2, c2cat)
    g216 = jnp.pad(cmax2, ((0, 0), (0, 0), (0, 8)))

    tab2f = tab2.reshape(B, 1, N * 8)
    acc2 = _sc_layer2(tab2f, g216, src_flat, dst_flat).reshape(B, 2, NP, 8)
    acc2 = jnp.transpose(acc2, (1, 0, 2, 3))

    psum, pcnt = _tc_norm2pool(acc2, tab2, cmax2, bias2[:, None, :],
                               batch.astype(jnp.int32)[:, None])

    return _tc_fin(psum, pcnt, Wf, bf)


# speedup vs baseline: 29.0604x; 1.0015x over previous
import functools

import jax
import jax.numpy as jnp
from jax import lax
from jax.experimental import pallas as pl
from jax.experimental.pallas import tpu as pltpu
from jax.experimental.pallas import tpu_sc as plsc

B = 4
N = 10000
E = 320000
D = 128
H = 4
F = 64
C = 4
G = 64

RT = 1000
NT = N // RT
CH = 80
CH1 = 128
EPTF = 19968
NCH1 = EPTF // CH1
EPT16 = E // 16
EPT32 = E // 32
NP = 10240
NPT = NP // 16
NH1 = 3456
NP1 = 3 * NH1
NH1T = NH1 // 16
NH = 5120
NHT = NH // 16


def _leaky(x):
    return jnp.where(x >= 0, x, 0.2 * x)


def _stats_body(x_ref, s_ref, q_ref):
    nt = pl.program_id(1)
    xb = x_ref[0]
    s = jnp.sum(xb, axis=0, keepdims=True)
    q = jnp.sum(xb * xb, axis=0, keepdims=True)

    @pl.when(nt == 0)
    def _():
        s_ref[0] = s
        q_ref[0] = q

    @pl.when(nt != 0)
    def _():
        s_ref[0] = s_ref[0] + s
        q_ref[0] = q_ref[0] + q


def _tc_stats(x, d):
    return pl.pallas_call(
        _stats_body,
        grid=(B, NT),
        in_specs=[pl.BlockSpec((1, RT, d), lambda b, nt: (b, nt, 0))],
        out_specs=[pl.BlockSpec((1, 1, d), lambda b, nt: (b, 0, 0)),
                   pl.BlockSpec((1, 1, d), lambda b, nt: (b, 0, 0))],
        out_shape=[jax.ShapeDtypeStruct((B, 1, d), jnp.float32),
                   jax.ShapeDtypeStruct((B, 1, d), jnp.float32)],
    )(x)


def _mm1_body(x_ref, w_ref, wa_ref, c1_ref, ca_ref,
              hp_ref, a1_ref, a2_ref, gmax_ref):
    nt = pl.program_id(1)
    xb = x_ref[0]
    h = jnp.dot(xb, w_ref[0], preferred_element_type=jnp.float32) + c1_ref[0]
    a = jnp.dot(xb, wa_ref[0], preferred_element_type=jnp.float32) + ca_ref[0]
    hp_ref[0, 0] = h[:, :128]
    hp_ref[0, 1] = h[:, 128:]
    a1_ref[0] = a
    a2_ref[0, 0] = jnp.concatenate([a[:, 0:2], a[:, 4:6]], axis=1)
    a2_ref[0, 1] = jnp.concatenate([a[:, 2:4], a[:, 6:8]], axis=1)
    cm = jnp.max(a, axis=0, keepdims=True)

    @pl.when(nt == 0)
    def _():
        gmax_ref[0] = cm

    @pl.when(nt != 0)
    def _():
        gmax_ref[0] = jnp.maximum(gmax_ref[0], cm)


def _tc_mm1(x, W1e, Wa, c1, ca):
    return pl.pallas_call(
        _mm1_body,
        grid=(B, NT),
        in_specs=[pl.BlockSpec((1, RT, D), lambda b, nt: (b, nt, 0)),
                  pl.BlockSpec((1, D, 256), lambda b, nt: (b, 0, 0)),
                  pl.BlockSpec((1, D, 8), lambda b, nt: (b, 0, 0)),
                  pl.BlockSpec((1, 1, 256), lambda b, nt: (b, 0, 0)),
                  pl.BlockSpec((1, 1, 8), lambda b, nt: (b, 0, 0))],
        out_specs=[pl.BlockSpec((1, 2, RT, 128), lambda b, nt: (b, 0, nt, 0)),
                   pl.BlockSpec((1, RT, 8), lambda b, nt: (b, nt, 0)),
                   pl.BlockSpec((1, 2, RT, 4), lambda b, nt: (b, 0, nt, 0)),
                   pl.BlockSpec((1, 1, 8), lambda b, nt: (b, 0, 0))],
        out_shape=[jax.ShapeDtypeStruct((B, 2, N, 128), jnp.float32),
                   jax.ShapeDtypeStruct((B, N, 8), jnp.float32),
                   jax.ShapeDtypeStruct((B, 2, N, 4), jnp.float32),
                   jax.ShapeDtypeStruct((B, 1, 8), jnp.float32)],
    )(x, W1e, Wa, c1, ca)


def _norm1_body(msg_ref, den_ref, hp_ref, a1_ref, gmax_ref, b1_ref,
                xr_ref, s2_ref, q2_ref):
    nt = pl.program_id(1)
    a = a1_ref[0]
    asrc = a[:, 0:4]
    adst = a[:, 4:8]
    gm = gmax_ref[0, 0, 0:4]
    shift = _leaky(gm[None, :] + adst)
    wself = jnp.exp(_leaky(asrc + adst) - shift)
    den4 = jnp.concatenate([den_ref[0, 0], den_ref[0, 1]], axis=1) + wself
    dinv = 1.0 / (den4 + 1e-16)
    for h in range(4):
        half = h // 2
        col = (h % 2) * 64
        msg = msg_ref[0, half][:, col:col + 64]
        hcols = hp_ref[0, half][:, col:col + 64]
        o = (msg + wself[:, h:h + 1] * hcols) * dinv[:, h:h + 1] \
            + b1_ref[0, 0, h * 64:(h + 1) * 64][None, :]
        xr = jnp.maximum(o, 0.0)
        xr_ref[0, :, h * 64:(h + 1) * 64] = xr
        s = jnp.sum(xr, axis=0, keepdims=True)
        q = jnp.sum(xr * xr, axis=0, keepdims=True)

        @pl.when(nt == 0)
        def _():
            s2_ref[0, :, h * 64:(h + 1) * 64] = s
            q2_ref[0, :, h * 64:(h + 1) * 64] = q

        @pl.when(nt != 0)
        def _():
            s2_ref[0, :, h * 64:(h + 1) * 64] = s2_ref[0, :, h * 64:(h + 1) * 64] + s
            q2_ref[0, :, h * 64:(h + 1) * 64] = q2_ref[0, :, h * 64:(h + 1) * 64] + q


def _tc_norm1(msgacc, den, hp, a1, gmax, bias1):
    return pl.pallas_call(
        _norm1_body,
        grid=(B, NT),
        in_specs=[pl.BlockSpec((1, 2, RT, 128), lambda b, nt: (b, 0, nt, 0)),
                  pl.BlockSpec((1, 2, RT, 2), lambda b, nt: (b, 0, nt, 0)),
                  pl.BlockSpec((1, 2, RT, 128), lambda b, nt: (b, 0, nt, 0)),
                  pl.BlockSpec((1, RT, 8), lambda b, nt: (b, nt, 0)),
                  pl.BlockSpec((1, 1, 8), lambda b, nt: (b, 0, 0)),
                  pl.BlockSpec((1, 1, 256), lambda b, nt: (b, 0, 0))],
        out_specs=[pl.BlockSpec((1, RT, 256), lambda b, nt: (b, nt, 0)),
                   pl.BlockSpec((1, 1, 256), lambda b, nt: (b, 0, 0)),
                   pl.BlockSpec((1, 1, 256), lambda b, nt: (b, 0, 0))],
        out_shape=[jax.ShapeDtypeStruct((B, N, 256), jnp.float32),
                   jax.ShapeDtypeStruct((B, 1, 256), jnp.float32),
                   jax.ShapeDtypeStruct((B, 1, 256), jnp.float32)],
    )(msgacc, den, hp, a1, gmax, bias1)


def _mm2_body(xr_ref, p2_ref, c2_ref, tab_ref, cmax_ref):
    nt = pl.program_id(1)
    t = jnp.dot(xr_ref[0], p2_ref[0], preferred_element_type=jnp.float32) + c2_ref[0]
    tab_ref[0] = t
    cm = jnp.max(t, axis=0, keepdims=True)

    @pl.when(nt == 0)
    def _():
        cmax_ref[0] = cm

    @pl.when(nt != 0)
    def _():
        cmax_ref[0] = jnp.maximum(cmax_ref[0], cm)


def _tc_mm2(xr, P2, c2):
    return pl.pallas_call(
        _mm2_body,
        grid=(B, NT),
        in_specs=[pl.BlockSpec((1, RT, 256), lambda b, nt: (b, nt, 0)),
                  pl.BlockSpec((1, 256, 8), lambda b, nt: (b, 0, 0)),
                  pl.BlockSpec((1, 1, 8), lambda b, nt: (b, 0, 0))],
        out_specs=[pl.BlockSpec((1, RT, 8), lambda b, nt: (b, nt, 0)),
                   pl.BlockSpec((1, 1, 8), lambda b, nt: (b, 0, 0))],
        out_shape=[jax.ShapeDtypeStruct((B, N, 8), jnp.float32),
                   jax.ShapeDtypeStruct((B, 1, 8), jnp.float32)],
    )(xr, P2, c2)


def _norm2_body(acc_ref, tab_ref, gmax_ref, b2_ref, batch_ref,
                ps_ref, pc_ref):
    nt = pl.program_id(1)
    th = tab_ref[0]
    h2 = th[:, 0:4]
    as2 = th[:, 4:5]
    ad2 = th[:, 5:6]
    g2 = gmax_ref[0, 0, 4]
    wself = jnp.exp(_leaky(as2 + ad2) - _leaky(g2 + ad2))
    a0 = acc_ref[0, 0]
    a1_ = acc_ref[1, 0]
    den = a0[:, 4:5] + a1_[:, 4:5] + wself
    msg = a0[:, 0:4] + a1_[:, 0:4] + wself * h2
    o2 = msg / (den + 1e-16) + b2_ref[0, 0][None, :]
    gel = 0.5 * o2 * (1.0 + lax.erf(o2 * 0.7071067811865476))
    oh = (batch_ref[:, 0:1] == lax.broadcasted_iota(jnp.int32, (1, G), 1)
          ).astype(jnp.float32)
    psum = lax.dot_general(oh, gel, (((0,), (0,)), ((), ())),
                           preferred_element_type=jnp.float32)
    ones = jnp.ones((RT, 1), jnp.float32)
    pcnt = lax.dot_general(oh, ones, (((0,), (0,)), ((), ())),
                           preferred_element_type=jnp.float32)

    @pl.when(nt == 0)
    def _():
        ps_ref[0] = psum
        pc_ref[0] = pcnt

    @pl.when(nt != 0)
    def _():
        ps_ref[0] = ps_ref[0] + psum
        pc_ref[0] = pc_ref[0] + pcnt


def _tc_norm2pool(acc2, tab2, gmax2, bias2, batch2d):
    return pl.pallas_call(
        _norm2_body,
        grid=(B, NT),
        in_specs=[pl.BlockSpec((2, 1, RT, 8), lambda b, nt: (0, b, nt, 0)),
                  pl.BlockSpec((1, RT, 8), lambda b, nt: (b, nt, 0)),
                  pl.BlockSpec((1, 1, 8), lambda b, nt: (b, 0, 0)),
                  pl.BlockSpec((1, 1, 4), lambda b, nt: (b, 0, 0)),
                  pl.BlockSpec((RT, 1), lambda b, nt: (nt, 0))],
        out_specs=[pl.BlockSpec((1, G, 4), lambda b, nt: (b, 0, 0)),
                   pl.BlockSpec((1, G, 1), lambda b, nt: (b, 0, 0))],
        out_shape=[jax.ShapeDtypeStruct((B, G, 4), jnp.float32),
                   jax.ShapeDtypeStruct((B, G, 1), jnp.float32)],
    )(acc2, tab2, gmax2, bias2, batch2d)


def _fin_body(ps_ref, pc_ref, wf_ref, bf_ref, out_ref):
    acc = jnp.zeros((G, C), jnp.float32)
    for b in range(B):
        pooled = ps_ref[b] / jnp.maximum(pc_ref[b], 1.0)
        m = jnp.max(pooled, axis=1, keepdims=True)
        z = pooled - m
        ls = z - jnp.log(jnp.sum(jnp.exp(z), axis=1, keepdims=True))
        acc = acc + jnp.dot(ls, wf_ref[4 * b:4 * b + 4, :],
                            preferred_element_type=jnp.float32)
    out_ref[...] = jnp.maximum(acc + bf_ref[0][None, :], 0.0)


def _tc_fin(psum, pcnt, Wf, bf):
    return pl.pallas_call(
        _fin_body,
        out_shape=jax.ShapeDtypeStruct((G, C), jnp.float32),
    )(psum, pcnt, Wf, bf[None, :])


def _sc1_body(a1_h, gmax_h, src_h, dst_h, hp_h,
              msg_h, den_h,
              tab_v, gbuf_v, idxs_v, idxd_v, idxl_v, iw_v, wab_v, wden_v,
              rows_v, zrow_v, zbuf_v, acc_s, dacc_s, sem, sem_s):
    c = lax.axis_index("c")
    sid = lax.axis_index("s")

    def zr(r, _):
        for q in range(8):
            zrow_v[r, pl.ds(q * 16, 16)] = jnp.zeros((16,), jnp.float32)
        return 0

    lax.fori_loop(0, 24, zr, 0)

    def zb(i, _):
        zbuf_v[pl.ds(i * 16, 16)] = jnp.zeros((16,), jnp.float32)
        return 0

    lax.fori_loop(0, 80, zb, 0)

    lo_ref = [None]

    def load_idx(base, ib):
        pltpu.sync_copy(src_h.at[pl.ds(base, CH1)], idxs_v.at[ib])
        pltpu.sync_copy(dst_h.at[pl.ds(base, CH1)], idxd_v.at[ib])

    def w_phase(pb, pz):
        lo = lo_ref[0]
        for j in range(CH1 // 16):
            sv = idxs_v[pb, pl.ds(j * 16, 16)]
            dv = idxd_v[pb, pl.ds(j * 16, 16)]
            inr = (dv >= lo) & (dv < lo + NH1)
            idxl_v[pb, pl.ds(j * 16, 16)] = jnp.where(inr, dv - lo, 0)
            for hh in range(2):
                hvec = jnp.full((16,), hh, jnp.int32)
                sa = plsc.load_gather(tab_v, [sv * 4 + hvec])
                da = plsc.load_gather(tab_v, [dv * 4 + hvec + 2])
                gm = plsc.load_gather(
                    gbuf_v, [jnp.broadcast_to(c * 2 + hh, (16,)).astype(jnp.int32)])
                w = jnp.exp(_leaky(sa + da) - _leaky(gm + da))
                wab_v[pl.ds(hh * CH1 + j * 16, 16)] = jnp.where(inr, w, 0.0)
                wden_v[pb, hh, pl.ds(j * 16, 16)] = jnp.where(pz, w, 0.0)
                iw_v[pb, hh, pl.ds(j * 16, 16)] = dv * 2 + hh

    def issue_gather(b, pb):
        pltpu.async_copy(hp_h.at[b, c].at[idxs_v.at[pb]], rows_v.at[pb], sem)

    def wait_gather(b, pb):
        pltpu.make_async_copy(hp_h.at[b, c].at[idxs_v.at[pb]],
                              rows_v.at[pb], sem).wait()

    def scale(pb):
        def sbody(e2, _):
            for u in range(4):
                e = e2 * 4 + u
                e16 = jnp.broadcast_to(e, (16,)).astype(jnp.int32)
                wa = plsc.load_gather(wab_v, [e16])
                wb = plsc.load_gather(wab_v, [e16 + CH1])
                for q in range(4):
                    rows_v[pb, e, pl.ds(q * 16, 16)] = \
                        rows_v[pb, e, pl.ds(q * 16, 16)] * wa
                for q in range(4, 8):
                    rows_v[pb, e, pl.ds(q * 16, 16)] = \
                        rows_v[pb, e, pl.ds(q * 16, 16)] * wb
            return 0

        lax.fori_loop(0, CH1 // 4, sbody, 0)

    def issue_scatter(pb):
        pltpu.async_copy(rows_v.at[pb], acc_s.at[idxl_v.at[pb]], sem_s, add=True)
        pltpu.async_copy(wden_v.at[pb, 0], dacc_s.at[iw_v.at[pb, 0]], sem_s, add=True)
        pltpu.async_copy(wden_v.at[pb, 1], dacc_s.at[iw_v.at[pb, 1]], sem_s, add=True)

    def wait_scatter(pb):
        pltpu.make_async_copy(rows_v.at[pb], acc_s.at[idxl_v.at[pb]], sem_s).wait()
        pltpu.make_async_copy(wden_v.at[pb, 0], dacc_s.at[iw_v.at[pb, 0]], sem_s).wait()
        pltpu.make_async_copy(wden_v.at[pb, 1], dacc_s.at[iw_v.at[pb, 1]], sem_s).wait()

    def branch_body(b, _):
        pltpu.sync_copy(zbuf_v, dacc_s.at[pl.ds(sid * 1280, 1280)])
        pltpu.sync_copy(a1_h.at[b, c, 0], tab_v)
        pltpu.sync_copy(gmax_h.at[b, 0], gbuf_v)

        def pass_body(p, _):
            lo_ref[0] = p * NH1
            pz = p == 0
            for i in range(9):
                pltpu.sync_copy(zrow_v,
                                acc_s.at[pl.ds(sid * NH1T + i * 24, 24)])
            plsc.subcore_barrier()

            tile0 = b * E + sid * EPTF
            load_idx(tile0, 0)
            issue_gather(b, 0)
            load_idx(tile0 + CH1, 1)
            w_phase(0, pz)
            issue_gather(b, 1)
            wait_gather(b, 0)
            scale(0)
            issue_scatter(0)

            def loop_body(k, _):
                pb = lax.rem(k, 2)
                load_idx(tile0 + (k + 1) * CH1, 1 - pb)
                w_phase(pb, pz)
                wait_scatter(1 - pb)
                issue_gather(b, 1 - pb)
                wait_gather(b, pb)
                scale(pb)
                issue_scatter(pb)
                return 0

            lax.fori_loop(1, NCH1, loop_body, 0)
            wait_scatter((NCH1 - 1) % 2)
            wait_gather(b, NCH1 % 2)

            @pl.when(sid < 4)
            def _():
                load_idx(b * E + 16 * EPTF + sid * CH1, 0)
                w_phase(0, pz)
                pltpu.async_copy(hp_h.at[b, c].at[idxs_v.at[0]],
                                 rows_v.at[0], sem).wait()
                scale(0)
                pltpu.sync_copy(rows_v.at[0], acc_s.at[idxl_v.at[0]], add=True)
                pltpu.sync_copy(wden_v.at[0, 0], dacc_s.at[iw_v.at[0, 0]], add=True)
                pltpu.sync_copy(wden_v.at[0, 1], dacc_s.at[iw_v.at[0, 1]], add=True)

            plsc.subcore_barrier()
            pltpu.sync_copy(acc_s.at[pl.ds(sid * NH1T, NH1T)],
                            msg_h.at[b, c, p, pl.ds(sid * NH1T, NH1T)])
            plsc.subcore_barrier()
            return 0

        lax.fori_loop(0, 3, pass_body, 0)

        @pl.when(sid == 0)
        def _():
            pltpu.sync_copy(dacc_s, den_h.at[b, c, 0])

        plsc.subcore_barrier()
        return 0

    lax.fori_loop(0, B, branch_body, 0)


def _sc_layer1(a1, gmax16, src_flat, dst_flat, hp):
    mesh = plsc.VectorSubcoreMesh(core_axis_name="c", subcore_axis_name="s")
    f = pl.kernel(
        _sc1_body,
        mesh=mesh,
        compiler_params=pltpu.CompilerParams(needs_layout_passes=False),
        out_type=[jax.ShapeDtypeStruct((B, 2, 3, NH1, 128), jnp.float32),
                  jax.ShapeDtypeStruct((B, 2, 1, 20480), jnp.float32)],
        scratch_types=[
            pltpu.VMEM((N * 4,), jnp.float32),
            pltpu.VMEM((16,), jnp.float32),
            pltpu.VMEM((2, CH1), jnp.int32),
            pltpu.VMEM((2, CH1), jnp.int32),
            pltpu.VMEM((2, CH1), jnp.int32),
            pltpu.VMEM((2, 2, CH1), jnp.int32),
            pltpu.VMEM((2 * CH1,), jnp.float32),
            pltpu.VMEM((2, 2, CH1), jnp.float32),
            pltpu.VMEM((2, CH1, 128), jnp.float32),
            pltpu.VMEM((24, 128), jnp.float32),
            pltpu.VMEM((1280,), jnp.float32),
            pltpu.VMEM_SHARED((NH1, 128), jnp.float32),
            pltpu.VMEM_SHARED((20480,), jnp.float32),
            pltpu.SemaphoreType.DMA,
            pltpu.SemaphoreType.DMA,
        ],
    )
    return f(a1, gmax16, src_flat, dst_flat, hp)


def _sc2_body(tab_h, gmax_h, src_h, dst_h,
              acc_h,
              tab_v, gbuf_v, idxs_v, idxd_v, i8_v, val_v, zbuf_v, acc8_s):
    c = lax.axis_index("c")
    sid = lax.axis_index("s")
    four16 = jnp.full((16,), 4, jnp.int32)
    five16 = jnp.full((16,), 5, jnp.int32)

    def zb(i, _):
        zbuf_v[pl.ds(i * 16, 16)] = jnp.zeros((16,), jnp.float32)
        return 0

    lax.fori_loop(0, 80, zb, 0)

    for b in range(B):
        pltpu.sync_copy(tab_h.at[b, 0], tab_v)
        pltpu.sync_copy(gmax_h.at[b, 0], gbuf_v)
        for p in range(2):
            lo = p * NH
            for i in range(2):
                pltpu.sync_copy(zbuf_v,
                                acc8_s.at[pl.ds(sid * 2560 + i * 1280, 1280)])
            plsc.subcore_barrier()

            def chunk(k, _):
                base = b * E + (sid * 2 + c) * EPT32 + k * CH
                pltpu.sync_copy(src_h.at[pl.ds(base, CH)], idxs_v)
                pltpu.sync_copy(dst_h.at[pl.ds(base, CH)], idxd_v)
                for j in range(CH // 16):
                    sv = idxs_v[pl.ds(j * 16, 16)]
                    dv = idxd_v[pl.ds(j * 16, 16)]
                    inr = (dv >= lo) & (dv < lo + NH)
                    dvl = jnp.where(inr, dv - lo, 0)
                    as2 = plsc.load_gather(tab_v, [sv * 8 + four16])
                    ad2 = plsc.load_gather(tab_v, [dv * 8 + five16])
                    g2 = plsc.load_gather(gbuf_v, [four16])
                    w = jnp.exp(_leaky(as2 + ad2) - _leaky(g2 + ad2))
                    w = jnp.where(inr, w, 0.0)
                    val_v[4, pl.ds(j * 16, 16)] = w
                    i8_v[4, pl.ds(j * 16, 16)] = dvl * 8 + 4
                    for cc in range(4):
                        cc16 = jnp.full((16,), cc, jnp.int32)
                        hcc = plsc.load_gather(tab_v, [sv * 8 + cc16])
                        val_v[cc, pl.ds(j * 16, 16)] = w * hcc
                        i8_v[cc, pl.ds(j * 16, 16)] = dvl * 8 + cc
                for cc in range(5):
                    pltpu.sync_copy(val_v.at[cc], acc8_s.at[i8_v.at[cc]], add=True)
                return 0

            lax.fori_loop(0, EPT32 // CH, chunk, 0)
            plsc.subcore_barrier()

            @pl.when(sid == 0)
            def _():
                pltpu.sync_copy(acc8_s, acc_h.at[b, c, p])

            plsc.subcore_barrier()


def _sc_layer2(tab2, gmax16, src_flat, dst_flat):
    mesh = plsc.VectorSubcoreMesh(core_axis_name="c", subcore_axis_name="s")
    f = pl.kernel(
        _sc2_body,
        mesh=mesh,
        compiler_params=pltpu.CompilerParams(needs_layout_passes=False),
        out_type=[jax.ShapeDtypeStruct((B, 2, 2, 8 * NH), jnp.float32)],
        scratch_types=[
            pltpu.VMEM((N * 8,), jnp.float32),
            pltpu.VMEM((16,), jnp.float32),
            pltpu.VMEM((CH,), jnp.int32),
            pltpu.VMEM((CH,), jnp.int32),
            pltpu.VMEM((5, CH), jnp.int32),
            pltpu.VMEM((5, CH), jnp.float32),
            pltpu.VMEM((1280,), jnp.float32),
            pltpu.VMEM_SHARED((8 * NH,), jnp.float32),
        ],
    )
    return f(tab2, gmax16, src_flat, dst_flat)[0]


def kernel(x, edge_index, batch, gamma1, beta1, W1, att_src1, att_dst1, bias1,
           gamma2, beta2, W2, att_src2, att_dst2, bias2, Wf, bf):
    ei = edge_index.astype(jnp.int32)

    s0, q0 = _tc_stats(x, D)
    mu = s0[:, 0, :] / N
    var = q0[:, 0, :] / N - mu * mu
    sc1 = gamma1 / jnp.sqrt(var + 1e-5)
    sh1 = beta1 - mu * sc1
    W1e = W1 * sc1[:, :, None]
    c1 = jnp.einsum("bd,bdk->bk", sh1, W1)[:, None, :]
    m4 = (jnp.arange(256)[:, None] // 64 == jnp.arange(4)[None, :]).astype(jnp.float32)
    A8 = jnp.concatenate([att_src1.reshape(B, 256, 1) * m4[None],
                          att_dst1.reshape(B, 256, 1) * m4[None]], axis=2)
    Wa = jnp.einsum("bdk,bkh->bdh", W1e, A8)
    ca = jnp.einsum("bk,bkh->bh", c1[:, 0, :], A8)[:, None, :]

    hp, a1, a2, gmax1 = _tc_mm1(x, W1e, Wa, c1, ca)
    g16 = jnp.pad(gmax1, ((0, 0), (0, 0), (0, 8)))
    src_flat = ei[:, 0, :].reshape(B * E)
    dst_flat = ei[:, 1, :].reshape(B * E)

    a2f = a2.reshape(B, 2, 1, N * 4)
    msgacc, denflat = _sc_layer1(a2f, g16, src_flat, dst_flat, hp)
    msgacc = msgacc.reshape(B, 2, NP1, 128)
    den = denflat[:, :, 0, :2 * N].reshape(B, 2, N, 2)

    xr, s2, q2 = _tc_norm1(msgacc, den, hp, a1, gmax1, bias1[:, None, :])
    mu2 = s2[:, 0, :] / N
    var2 = q2[:, 0, :] / N - mu2 * mu2
    sc2 = gamma2 / jnp.sqrt(var2 + 1e-5)
    sh2 = beta2 - mu2 * sc2
    W2e = W2 * sc2[:, :, None]
    c2 = jnp.einsum("bd,bdk->bk", sh2, W2)
    as2v = att_src2[:, 0, :]
    ad2v = att_dst2[:, 0, :]
    P2 = jnp.concatenate([
        W2e,
        jnp.einsum("bdk,bk->bd", W2e, as2v)[:, :, None],
        jnp.einsum("bdk,bk->bd", W2e, ad2v)[:, :, None],
        jnp.zeros((B, 256, 2), jnp.float32),
    ], axis=2)
    c2cat = jnp.concatenate([
        c2,
        jnp.einsum("bk,bk->b", c2, as2v)[:, None],
        jnp.einsum("bk,bk->b", c2, ad2v)[:, None],
        jnp.zeros((B, 2), jnp.float32),
    ], axis=1)[:, None, :]

    tab2, cmax2 = _tc_mm2(xr, P2, c2cat)
    g216 = jnp.pad(cmax2, ((0, 0), (0, 0), (0, 8)))

    tab2f = tab2.reshape(B, 1, N * 8)
    acc2 = _sc_layer2(tab2f, g216, src_flat, dst_flat).reshape(B, 2, NP, 8)
    acc2 = jnp.transpose(acc2, (1, 0, 2, 3))

    psum, pcnt = _tc_norm2pool(acc2, tab2, cmax2, bias2[:, None, :],
                               batch.astype(jnp.int32)[:, None])

    return _tc_fin(psum, pcnt, Wf, bf)


# speedup vs baseline: 34.6222x; 1.1914x over previous
import functools

import jax
import jax.numpy as jnp
from jax import lax
from jax.experimental import pallas as pl
from jax.experimental.pallas import tpu as pltpu
from jax.experimental.pallas import tpu_sc as plsc

B = 4
N = 10000
E = 320000
D = 128
H = 4
F = 64
C = 4
G = 64

RT = 1000
NT = N // RT
CH = 80
CH1 = 128
EPTF = 19968
NCH1 = EPTF // CH1
EPT16 = E // 16
EPT32 = E // 32
NP = 10240
NPT = NP // 16
NH1 = 3456
NP1 = 3 * NH1
NH1T = NH1 // 16
NH = 5120
NHT = NH // 16


def _leaky(x):
    return jnp.where(x >= 0, x, 0.2 * x)


def _stats_body(x_ref, s_ref, q_ref):
    nt = pl.program_id(1)
    xb = x_ref[0]
    s = jnp.sum(xb, axis=0, keepdims=True)
    q = jnp.sum(xb * xb, axis=0, keepdims=True)

    @pl.when(nt == 0)
    def _():
        s_ref[0] = s
        q_ref[0] = q

    @pl.when(nt != 0)
    def _():
        s_ref[0] = s_ref[0] + s
        q_ref[0] = q_ref[0] + q


def _tc_stats(x, d):
    return pl.pallas_call(
        _stats_body,
        grid=(B, NT),
        in_specs=[pl.BlockSpec((1, RT, d), lambda b, nt: (b, nt, 0))],
        out_specs=[pl.BlockSpec((1, 1, d), lambda b, nt: (b, 0, 0)),
                   pl.BlockSpec((1, 1, d), lambda b, nt: (b, 0, 0))],
        out_shape=[jax.ShapeDtypeStruct((B, 1, d), jnp.float32),
                   jax.ShapeDtypeStruct((B, 1, d), jnp.float32)],
    )(x)


def _mm1_body(x_ref, w_ref, wa_ref, c1_ref, ca_ref,
              hp_ref, a1_ref, a2_ref, gmax_ref):
    nt = pl.program_id(1)
    xb = x_ref[0]
    h = jnp.dot(xb, w_ref[0], preferred_element_type=jnp.float32) + c1_ref[0]
    a = jnp.dot(xb, wa_ref[0], preferred_element_type=jnp.float32) + ca_ref[0]
    hp_ref[0, 0] = h[:, :128]
    hp_ref[0, 1] = h[:, 128:]
    a1_ref[0] = a
    a2_ref[0, 0] = jnp.concatenate([a[:, 0:2], a[:, 4:6]], axis=1)
    a2_ref[0, 1] = jnp.concatenate([a[:, 2:4], a[:, 6:8]], axis=1)
    cm = jnp.max(a, axis=0, keepdims=True)

    @pl.when(nt == 0)
    def _():
        gmax_ref[0] = cm

    @pl.when(nt != 0)
    def _():
        gmax_ref[0] = jnp.maximum(gmax_ref[0], cm)


def _tc_mm1(x, W1e, Wa, c1, ca):
    return pl.pallas_call(
        _mm1_body,
        grid=(B, NT),
        in_specs=[pl.BlockSpec((1, RT, D), lambda b, nt: (b, nt, 0)),
                  pl.BlockSpec((1, D, 256), lambda b, nt: (b, 0, 0)),
                  pl.BlockSpec((1, D, 8), lambda b, nt: (b, 0, 0)),
                  pl.BlockSpec((1, 1, 256), lambda b, nt: (b, 0, 0)),
                  pl.BlockSpec((1, 1, 8), lambda b, nt: (b, 0, 0))],
        out_specs=[pl.BlockSpec((1, 2, RT, 128), lambda b, nt: (b, 0, nt, 0)),
                   pl.BlockSpec((1, RT, 8), lambda b, nt: (b, nt, 0)),
                   pl.BlockSpec((1, 2, RT, 4), lambda b, nt: (b, 0, nt, 0)),
                   pl.BlockSpec((1, 1, 8), lambda b, nt: (b, 0, 0))],
        out_shape=[jax.ShapeDtypeStruct((B, 2, N, 128), jnp.float32),
                   jax.ShapeDtypeStruct((B, N, 8), jnp.float32),
                   jax.ShapeDtypeStruct((B, 2, N, 4), jnp.float32),
                   jax.ShapeDtypeStruct((B, 1, 8), jnp.float32)],
    )(x, W1e, Wa, c1, ca)


def _norm1_body(msg_ref, den_ref, hp_ref, a1_ref, gmax_ref, b1_ref,
                xr_ref, s2_ref, q2_ref):
    nt = pl.program_id(1)
    a = a1_ref[0]
    asrc = a[:, 0:4]
    adst = a[:, 4:8]
    gm = gmax_ref[0, 0, 0:4]
    shift = _leaky(gm[None, :] + adst)
    wself = jnp.exp(_leaky(asrc + adst) - shift)
    den4 = jnp.concatenate([den_ref[0, 0], den_ref[0, 1]], axis=1) + wself
    dinv = 1.0 / (den4 + 1e-16)
    for h in range(4):
        half = h // 2
        col = (h % 2) * 64
        msg = msg_ref[0, half][:, col:col + 64]
        hcols = hp_ref[0, half][:, col:col + 64]
        o = (msg + wself[:, h:h + 1] * hcols) * dinv[:, h:h + 1] \
            + b1_ref[0, 0, h * 64:(h + 1) * 64][None, :]
        xr = jnp.maximum(o, 0.0)
        xr_ref[0, :, h * 64:(h + 1) * 64] = xr
        s = jnp.sum(xr, axis=0, keepdims=True)
        q = jnp.sum(xr * xr, axis=0, keepdims=True)

        @pl.when(nt == 0)
        def _():
            s2_ref[0, :, h * 64:(h + 1) * 64] = s
            q2_ref[0, :, h * 64:(h + 1) * 64] = q

        @pl.when(nt != 0)
        def _():
            s2_ref[0, :, h * 64:(h + 1) * 64] = s2_ref[0, :, h * 64:(h + 1) * 64] + s
            q2_ref[0, :, h * 64:(h + 1) * 64] = q2_ref[0, :, h * 64:(h + 1) * 64] + q


def _tc_norm1(msgacc, den, hp, a1, gmax, bias1):
    return pl.pallas_call(
        _norm1_body,
        grid=(B, NT),
        in_specs=[pl.BlockSpec((1, 2, RT, 128), lambda b, nt: (b, 0, nt, 0)),
                  pl.BlockSpec((1, 2, RT, 2), lambda b, nt: (b, 0, nt, 0)),
                  pl.BlockSpec((1, 2, RT, 128), lambda b, nt: (b, 0, nt, 0)),
                  pl.BlockSpec((1, RT, 8), lambda b, nt: (b, nt, 0)),
                  pl.BlockSpec((1, 1, 8), lambda b, nt: (b, 0, 0)),
                  pl.BlockSpec((1, 1, 256), lambda b, nt: (b, 0, 0))],
        out_specs=[pl.BlockSpec((1, RT, 256), lambda b, nt: (b, nt, 0)),
                   pl.BlockSpec((1, 1, 256), lambda b, nt: (b, 0, 0)),
                   pl.BlockSpec((1, 1, 256), lambda b, nt: (b, 0, 0))],
        out_shape=[jax.ShapeDtypeStruct((B, N, 256), jnp.float32),
                   jax.ShapeDtypeStruct((B, 1, 256), jnp.float32),
                   jax.ShapeDtypeStruct((B, 1, 256), jnp.float32)],
    )(msgacc, den, hp, a1, gmax, bias1)


def _mm2_body(xr_ref, p2_ref, c2_ref, tab_ref, cmax_ref):
    nt = pl.program_id(1)
    t = jnp.dot(xr_ref[0], p2_ref[0], preferred_element_type=jnp.float32) + c2_ref[0]
    tab_ref[0] = t
    cm = jnp.max(t, axis=0, keepdims=True)

    @pl.when(nt == 0)
    def _():
        cmax_ref[0] = cm

    @pl.when(nt != 0)
    def _():
        cmax_ref[0] = jnp.maximum(cmax_ref[0], cm)


def _tc_mm2(xr, P2, c2):
    return pl.pallas_call(
        _mm2_body,
        grid=(B, NT),
        in_specs=[pl.BlockSpec((1, RT, 256), lambda b, nt: (b, nt, 0)),
                  pl.BlockSpec((1, 256, 8), lambda b, nt: (b, 0, 0)),
                  pl.BlockSpec((1, 1, 8), lambda b, nt: (b, 0, 0))],
        out_specs=[pl.BlockSpec((1, RT, 8), lambda b, nt: (b, nt, 0)),
                   pl.BlockSpec((1, 1, 8), lambda b, nt: (b, 0, 0))],
        out_shape=[jax.ShapeDtypeStruct((B, N, 8), jnp.float32),
                   jax.ShapeDtypeStruct((B, 1, 8), jnp.float32)],
    )(xr, P2, c2)


def _norm2_body(acc_ref, tab_ref, gmax_ref, b2_ref, batch_ref,
                ps_ref, pc_ref):
    nt = pl.program_id(1)
    th = tab_ref[0]
    h2 = th[:, 0:4]
    as2 = th[:, 4:5]
    ad2 = th[:, 5:6]
    g2 = gmax_ref[0, 0, 4]
    wself = jnp.exp(_leaky(as2 + ad2) - _leaky(g2 + ad2))
    a0 = acc_ref[0, 0]
    a1_ = acc_ref[1, 0]
    den = a0[:, 4:5] + a1_[:, 4:5] + wself
    msg = a0[:, 0:4] + a1_[:, 0:4] + wself * h2
    o2 = msg / (den + 1e-16) + b2_ref[0, 0][None, :]
    gel = 0.5 * o2 * (1.0 + lax.erf(o2 * 0.7071067811865476))
    oh = (batch_ref[:, 0:1] == lax.broadcasted_iota(jnp.int32, (1, G), 1)
          ).astype(jnp.float32)
    psum = lax.dot_general(oh, gel, (((0,), (0,)), ((), ())),
                           preferred_element_type=jnp.float32)
    ones = jnp.ones((RT, 1), jnp.float32)
    pcnt = lax.dot_general(oh, ones, (((0,), (0,)), ((), ())),
                           preferred_element_type=jnp.float32)

    @pl.when(nt == 0)
    def _():
        ps_ref[0] = psum
        pc_ref[0] = pcnt

    @pl.when(nt != 0)
    def _():
        ps_ref[0] = ps_ref[0] + psum
        pc_ref[0] = pc_ref[0] + pcnt


def _tc_norm2pool(acc2, tab2, gmax2, bias2, batch2d):
    return pl.pallas_call(
        _norm2_body,
        grid=(B, NT),
        in_specs=[pl.BlockSpec((2, 1, RT, 8), lambda b, nt: (0, b, nt, 0)),
                  pl.BlockSpec((1, RT, 8), lambda b, nt: (b, nt, 0)),
                  pl.BlockSpec((1, 1, 8), lambda b, nt: (b, 0, 0)),
                  pl.BlockSpec((1, 1, 4), lambda b, nt: (b, 0, 0)),
                  pl.BlockSpec((RT, 1), lambda b, nt: (nt, 0))],
        out_specs=[pl.BlockSpec((1, G, 4), lambda b, nt: (b, 0, 0)),
                   pl.BlockSpec((1, G, 1), lambda b, nt: (b, 0, 0))],
        out_shape=[jax.ShapeDtypeStruct((B, G, 4), jnp.float32),
                   jax.ShapeDtypeStruct((B, G, 1), jnp.float32)],
    )(acc2, tab2, gmax2, bias2, batch2d)


def _fin_body(ps_ref, pc_ref, wf_ref, bf_ref, out_ref):
    acc = jnp.zeros((G, C), jnp.float32)
    for b in range(B):
        pooled = ps_ref[b] / jnp.maximum(pc_ref[b], 1.0)
        m = jnp.max(pooled, axis=1, keepdims=True)
        z = pooled - m
        ls = z - jnp.log(jnp.sum(jnp.exp(z), axis=1, keepdims=True))
        acc = acc + jnp.dot(ls, wf_ref[4 * b:4 * b + 4, :],
                            preferred_element_type=jnp.float32)
    out_ref[...] = jnp.maximum(acc + bf_ref[0][None, :], 0.0)


def _tc_fin(psum, pcnt, Wf, bf):
    return pl.pallas_call(
        _fin_body,
        out_shape=jax.ShapeDtypeStruct((G, C), jnp.float32),
    )(psum, pcnt, Wf, bf[None, :])


def _sc1_body(a1_h, gmax_h, src_h, dst_h, hp_h,
              msg_h, den_h,
              tab_v, gbuf_v, idxs_v, idxd_v, idxl_v, iw_v, wab_v, wden_v,
              rows_v, zrow_v, zbuf_v, acc_s, dacc_s, sem, sem_s):
    c = lax.axis_index("c")
    sid = lax.axis_index("s")

    def zr(r, _):
        for q in range(8):
            zrow_v[r, pl.ds(q * 16, 16)] = jnp.zeros((16,), jnp.float32)
        return 0

    lax.fori_loop(0, 24, zr, 0)

    def zb(i, _):
        zbuf_v[pl.ds(i * 16, 16)] = jnp.zeros((16,), jnp.float32)
        return 0

    lax.fori_loop(0, 80, zb, 0)

    lo_ref = [None]

    def load_idx(base, ib):
        pltpu.sync_copy(src_h.at[pl.ds(base, CH1)], idxs_v.at[ib])
        pltpu.sync_copy(dst_h.at[pl.ds(base, CH1)], idxd_v.at[ib])

    def w_phase(pb, pz):
        lo = lo_ref[0]
        for j in range(CH1 // 16):
            sv = idxs_v[pb, pl.ds(j * 16, 16)]
            dv = idxd_v[pb, pl.ds(j * 16, 16)]
            inr = (dv >= lo) & (dv < lo + NH1)
            idxl_v[pb, pl.ds(j * 16, 16)] = jnp.where(inr, dv - lo, 0)
            for hh in range(2):
                hvec = jnp.full((16,), hh, jnp.int32)
                sa = plsc.load_gather(tab_v, [sv * 4 + hvec])
                da = plsc.load_gather(tab_v, [dv * 4 + hvec + 2])
                gm = plsc.load_gather(
                    gbuf_v, [jnp.broadcast_to(c * 2 + hh, (16,)).astype(jnp.int32)])
                w = jnp.exp(_leaky(sa + da) - _leaky(gm + da))
                wab_v[pl.ds(hh * CH1 + j * 16, 16)] = jnp.where(inr, w, 0.0)
                wden_v[pb, hh, pl.ds(j * 16, 16)] = jnp.where(pz, w, 0.0)
                iw_v[pb, hh, pl.ds(j * 16, 16)] = dv * 2 + hh

    def issue_gather(b, pb):
        pltpu.async_copy(hp_h.at[b, c].at[idxs_v.at[pb]], rows_v.at[pb], sem)

    def wait_gather(b, pb):
        pltpu.make_async_copy(hp_h.at[b, c].at[idxs_v.at[pb]],
                              rows_v.at[pb], sem).wait()

    def scale(pb):
        def sbody(e2, _):
            for u in range(4):
                e = e2 * 4 + u
                e16 = jnp.broadcast_to(e, (16,)).astype(jnp.int32)
                wa = plsc.load_gather(wab_v, [e16])
                wb = plsc.load_gather(wab_v, [e16 + CH1])
                for q in range(4):
                    rows_v[pb, e, pl.ds(q * 16, 16)] = \
                        rows_v[pb, e, pl.ds(q * 16, 16)] * wa
                for q in range(4, 8):
                    rows_v[pb, e, pl.ds(q * 16, 16)] = \
                        rows_v[pb, e, pl.ds(q * 16, 16)] * wb
            return 0

        lax.fori_loop(0, CH1 // 4, sbody, 0)

    def issue_scatter(pb):
        pltpu.async_copy(rows_v.at[pb], acc_s.at[idxl_v.at[pb]], sem_s, add=True)
        pltpu.async_copy(wden_v.at[pb, 0], dacc_s.at[iw_v.at[pb, 0]], sem_s, add=True)
        pltpu.async_copy(wden_v.at[pb, 1], dacc_s.at[iw_v.at[pb, 1]], sem_s, add=True)

    def wait_scatter(pb):
        pltpu.make_async_copy(rows_v.at[pb], acc_s.at[idxl_v.at[pb]], sem_s).wait()
        pltpu.make_async_copy(wden_v.at[pb, 0], dacc_s.at[iw_v.at[pb, 0]], sem_s).wait()
        pltpu.make_async_copy(wden_v.at[pb, 1], dacc_s.at[iw_v.at[pb, 1]], sem_s).wait()

    def branch_body(b, _):
        pltpu.sync_copy(zbuf_v, dacc_s.at[pl.ds(sid * 1280, 1280)])
        pltpu.sync_copy(a1_h.at[b, c, 0], tab_v)
        pltpu.sync_copy(gmax_h.at[b, 0], gbuf_v)

        def pass_body(p, _):
            lo_ref[0] = p * NH1
            pz = p == 0
            for i in range(9):
                pltpu.sync_copy(zrow_v,
                                acc_s.at[pl.ds(sid * NH1T + i * 24, 24)])
            plsc.subcore_barrier()

            tile0 = b * E + sid * EPTF
            load_idx(tile0, 0)
            issue_gather(b, 0)
            load_idx(tile0 + CH1, 1)
            w_phase(0, pz)
            issue_gather(b, 1)
            wait_gather(b, 0)
            scale(0)
            issue_scatter(0)

            def loop_body(k, _):
                pb = lax.rem(k, 2)
                load_idx(tile0 + (k + 1) * CH1, 1 - pb)
                w_phase(pb, pz)
                wait_scatter(1 - pb)
                issue_gather(b, 1 - pb)
                wait_gather(b, pb)
                scale(pb)
                issue_scatter(pb)
                return 0

            lax.fori_loop(1, NCH1, loop_body, 0)
            wait_scatter((NCH1 - 1) % 2)
            wait_gather(b, NCH1 % 2)

            @pl.when(sid < 4)
            def _():
                load_idx(b * E + 16 * EPTF + sid * CH1, 0)
                w_phase(0, pz)
                pltpu.async_copy(hp_h.at[b, c].at[idxs_v.at[0]],
                                 rows_v.at[0], sem).wait()
                scale(0)
                pltpu.sync_copy(rows_v.at[0], acc_s.at[idxl_v.at[0]], add=True)
                pltpu.sync_copy(wden_v.at[0, 0], dacc_s.at[iw_v.at[0, 0]], add=True)
                pltpu.sync_copy(wden_v.at[0, 1], dacc_s.at[iw_v.at[0, 1]], add=True)

            plsc.subcore_barrier()
            pltpu.sync_copy(acc_s.at[pl.ds(sid * NH1T, NH1T)],
                            msg_h.at[b, c, p, pl.ds(sid * NH1T, NH1T)])
            plsc.subcore_barrier()
            return 0

        lax.fori_loop(0, 3, pass_body, 0)

        @pl.when(sid == 0)
        def _():
            pltpu.sync_copy(dacc_s, den_h.at[b, c, 0])

        plsc.subcore_barrier()
        return 0

    lax.fori_loop(0, B, branch_body, 0)


def _sc_layer1(a1, gmax16, src_flat, dst_flat, hp):
    mesh = plsc.VectorSubcoreMesh(core_axis_name="c", subcore_axis_name="s")
    f = pl.kernel(
        _sc1_body,
        mesh=mesh,
        compiler_params=pltpu.CompilerParams(needs_layout_passes=False),
        out_type=[jax.ShapeDtypeStruct((B, 2, 3, NH1, 128), jnp.float32),
                  jax.ShapeDtypeStruct((B, 2, 1, 20480), jnp.float32)],
        scratch_types=[
            pltpu.VMEM((N * 4,), jnp.float32),
            pltpu.VMEM((16,), jnp.float32),
            pltpu.VMEM((2, CH1), jnp.int32),
            pltpu.VMEM((2, CH1), jnp.int32),
            pltpu.VMEM((2, CH1), jnp.int32),
            pltpu.VMEM((2, 2, CH1), jnp.int32),
            pltpu.VMEM((2 * CH1,), jnp.float32),
            pltpu.VMEM((2, 2, CH1), jnp.float32),
            pltpu.VMEM((2, CH1, 128), jnp.float32),
            pltpu.VMEM((24, 128), jnp.float32),
            pltpu.VMEM((1280,), jnp.float32),
            pltpu.VMEM_SHARED((NH1, 128), jnp.float32),
            pltpu.VMEM_SHARED((20480,), jnp.float32),
            pltpu.SemaphoreType.DMA,
            pltpu.SemaphoreType.DMA,
        ],
    )
    return f(a1, gmax16, src_flat, dst_flat, hp)


def _sc2_body(tab_h, gmax_h, src_h, dst_h,
              acc_h,
              tab_v, gbuf_v, idxs_v, idxd_v, i8_v, val_v, zbuf_v, acc8_s, sem_s):
    c = lax.axis_index("c")
    sid = lax.axis_index("s")
    wid = sid * 2 + c
    four16 = jnp.full((16,), 4, jnp.int32)
    five16 = jnp.full((16,), 5, jnp.int32)

    def zb(i, _):
        zbuf_v[pl.ds(i * 16, 16)] = jnp.zeros((16,), jnp.float32)
        return 0

    lax.fori_loop(0, 80, zb, 0)

    def load_idx(base, ib):
        pltpu.sync_copy(src_h.at[pl.ds(base, CH1)], idxs_v.at[ib])
        pltpu.sync_copy(dst_h.at[pl.ds(base, CH1)], idxd_v.at[ib])

    def compute(pb):
        for j in range(CH1 // 16):
            sv = idxs_v[pb, pl.ds(j * 16, 16)]
            dv = idxd_v[pb, pl.ds(j * 16, 16)]
            as2 = plsc.load_gather(tab_v, [sv * 8 + four16])
            ad2 = plsc.load_gather(tab_v, [dv * 8 + five16])
            g2 = plsc.load_gather(gbuf_v, [four16])
            w = jnp.exp(_leaky(as2 + ad2) - _leaky(g2 + ad2))
            val_v[pb, 4, pl.ds(j * 16, 16)] = w
            i8_v[pb, 4, pl.ds(j * 16, 16)] = dv * 8 + 4
            for cc in range(4):
                cc16 = jnp.full((16,), cc, jnp.int32)
                hcc = plsc.load_gather(tab_v, [sv * 8 + cc16])
                val_v[pb, cc, pl.ds(j * 16, 16)] = w * hcc
                i8_v[pb, cc, pl.ds(j * 16, 16)] = dv * 8 + cc

    def issue_scatter(pb):
        for cc in range(5):
            pltpu.async_copy(val_v.at[pb, cc], acc8_s.at[i8_v.at[pb, cc]],
                             sem_s, add=True)

    def wait_scatter(pb):
        for cc in range(5):
            pltpu.make_async_copy(val_v.at[pb, cc], acc8_s.at[i8_v.at[pb, cc]],
                                  sem_s).wait()

    def branch_body(b, _):
        pltpu.sync_copy(tab_h.at[b, 0], tab_v)
        pltpu.sync_copy(gmax_h.at[b, 0], gbuf_v)
        for i in range(4):
            pltpu.sync_copy(zbuf_v,
                            acc8_s.at[pl.ds(sid * 5120 + i * 1280, 1280)])
        plsc.subcore_barrier()

        tile0 = b * E + wid * 9984
        load_idx(tile0, 0)
        compute(0)
        issue_scatter(0)

        def loop_body(k, _):
            pb = lax.rem(k, 2)
            load_idx(tile0 + k * CH1, pb)
            compute(pb)
            wait_scatter(1 - pb)
            issue_scatter(pb)
            return 0

        lax.fori_loop(1, 78, loop_body, 0)
        wait_scatter(77 % 2)

        @pl.when(wid < 4)
        def _():
            load_idx(b * E + 32 * 9984 + wid * CH1, 0)
            compute(0)
            for cc in range(5):
                pltpu.sync_copy(val_v.at[0, cc], acc8_s.at[i8_v.at[0, cc]],
                                add=True)

        plsc.subcore_barrier()

        @pl.when(sid == 0)
        def _():
            pltpu.sync_copy(acc8_s, acc_h.at[b, c, 0])

        plsc.subcore_barrier()
        return 0

    lax.fori_loop(0, B, branch_body, 0)


def _sc_layer2(tab2, gmax16, src_flat, dst_flat):
    mesh = plsc.VectorSubcoreMesh(core_axis_name="c", subcore_axis_name="s")
    f = pl.kernel(
        _sc2_body,
        mesh=mesh,
        compiler_params=pltpu.CompilerParams(needs_layout_passes=False),
        out_type=[jax.ShapeDtypeStruct((B, 2, 1, 81920), jnp.float32)],
        scratch_types=[
            pltpu.VMEM((N * 8,), jnp.float32),
            pltpu.VMEM((16,), jnp.float32),
            pltpu.VMEM((2, CH1), jnp.int32),
            pltpu.VMEM((2, CH1), jnp.int32),
            pltpu.VMEM((2, 8, CH1), jnp.int32),
            pltpu.VMEM((2, 8, CH1), jnp.float32),
            pltpu.VMEM((1280,), jnp.float32),
            pltpu.VMEM_SHARED((81920,), jnp.float32),
            pltpu.SemaphoreType.DMA,
        ],
    )
    return f(tab2, gmax16, src_flat, dst_flat)[0]


def kernel(x, edge_index, batch, gamma1, beta1, W1, att_src1, att_dst1, bias1,
           gamma2, beta2, W2, att_src2, att_dst2, bias2, Wf, bf):
    ei = edge_index.astype(jnp.int32)

    s0, q0 = _tc_stats(x, D)
    mu = s0[:, 0, :] / N
    var = q0[:, 0, :] / N - mu * mu
    sc1 = gamma1 / jnp.sqrt(var + 1e-5)
    sh1 = beta1 - mu * sc1
    W1e = W1 * sc1[:, :, None]
    c1 = jnp.einsum("bd,bdk->bk", sh1, W1)[:, None, :]
    m4 = (jnp.arange(256)[:, None] // 64 == jnp.arange(4)[None, :]).astype(jnp.float32)
    A8 = jnp.concatenate([att_src1.reshape(B, 256, 1) * m4[None],
                          att_dst1.reshape(B, 256, 1) * m4[None]], axis=2)
    Wa = jnp.einsum("bdk,bkh->bdh", W1e, A8)
    ca = jnp.einsum("bk,bkh->bh", c1[:, 0, :], A8)[:, None, :]

    hp, a1, a2, gmax1 = _tc_mm1(x, W1e, Wa, c1, ca)
    g16 = jnp.pad(gmax1, ((0, 0), (0, 0), (0, 8)))
    src_flat = ei[:, 0, :].reshape(B * E)
    dst_flat = ei[:, 1, :].reshape(B * E)

    a2f = a2.reshape(B, 2, 1, N * 4)
    msgacc, denflat = _sc_layer1(a2f, g16, src_flat, dst_flat, hp)
    msgacc = msgacc.reshape(B, 2, NP1, 128)
    den = denflat[:, :, 0, :2 * N].reshape(B, 2, N, 2)

    xr, s2, q2 = _tc_norm1(msgacc, den, hp, a1, gmax1, bias1[:, None, :])
    mu2 = s2[:, 0, :] / N
    var2 = q2[:, 0, :] / N - mu2 * mu2
    sc2 = gamma2 / jnp.sqrt(var2 + 1e-5)
    sh2 = beta2 - mu2 * sc2
    W2e = W2 * sc2[:, :, None]
    c2 = jnp.einsum("bd,bdk->bk", sh2, W2)
    as2v = att_src2[:, 0, :]
    ad2v = att_dst2[:, 0, :]
    P2 = jnp.concatenate([
        W2e,
        jnp.einsum("bdk,bk->bd", W2e, as2v)[:, :, None],
        jnp.einsum("bdk,bk->bd", W2e, ad2v)[:, :, None],
        jnp.zeros((B, 256, 2), jnp.float32),
    ], axis=2)
    c2cat = jnp.concatenate([
        c2,
        jnp.einsum("bk,bk->b", c2, as2v)[:, None],
        jnp.einsum("bk,bk->b", c2, ad2v)[:, None],
        jnp.zeros((B, 2), jnp.float32),
    ], axis=1)[:, None, :]

    tab2, cmax2 = _tc_mm2(xr, P2, c2cat)
    g216 = jnp.pad(cmax2, ((0, 0), (0, 0), (0, 8)))

    tab2f = tab2.reshape(B, 1, N * 8)
    acc2 = _sc_layer2(tab2f, g216, src_flat, dst_flat)[:, :, 0, :8 * N].reshape(B, 2, N, 8)
    acc2 = jnp.transpose(acc2, (1, 0, 2, 3))

    psum, pcnt = _tc_norm2pool(acc2, tab2, cmax2, bias2[:, None, :],
                               batch.astype(jnp.int32)[:, None])

    return _tc_fin(psum, pcnt, Wf, bf)
